# baseline (speedup 1.0000x reference)
import functools

import numpy as np
import jax
import jax.numpy as jnp
from jax import lax
from jax.experimental import pallas as pl
from jax.experimental.pallas import tpu as pltpu
from jax.experimental.pallas import tpu_sc as plsc

POS_THRESH = 0.1
NEG_THRESH = 1.4
NUM_POS = 5192
NUM_HN_SAMPLES = 2048

NW = 32
NS = 16
PCH = 88
NPCH = 2
PB = PCH * NPCH
M_PAD = NW * PB
BLK_M = 1408
KROWS = 640
KBUF = KROWS + 32
CK_PAD = NS * KROWS + 512
KCHUNK = 512
SENT = 0x7FFFFFFF


@functools.lru_cache(maxsize=None)
def _selections(N0, N1, n_pairs):
    rng = np.random.RandomState(0)
    sel0 = rng.choice(N0, min(N0, NUM_HN_SAMPLES), replace=False)
    sel1 = rng.choice(N1, min(N1, NUM_HN_SAMPLES), replace=False)
    if n_pairs > NUM_POS:
        pos_sel = rng.choice(n_pairs, NUM_POS, replace=False)
    else:
        pos_sel = np.arange(n_pairs)
    return sel0.astype(np.int32), sel1.astype(np.int32), pos_sel.astype(np.int32)


def _sc_gather_fn(n_sub, n_pairs):
    sb = n_sub // NW
    mesh = plsc.VectorSubcoreMesh(core_axis_name="c", subcore_axis_name="s")
    out_type = [
        jax.ShapeDtypeStruct((M_PAD, 128), jnp.float32),
        jax.ShapeDtypeStruct((M_PAD, 128), jnp.float32),
        jax.ShapeDtypeStruct((n_sub, 128), jnp.float32),
        jax.ShapeDtypeStruct((n_sub, 128), jnp.float32),
        jax.ShapeDtypeStruct((NW, NPCH, PCH), jnp.int32),
        jax.ShapeDtypeStruct((NW, NPCH, PCH), jnp.int32),
        jax.ShapeDtypeStruct((CK_PAD,), jnp.int32),
        jax.ShapeDtypeStruct((CK_PAD,), jnp.int32),
        jax.ShapeDtypeStruct((2, 16), jnp.int32),
    ]
    scratch = [
        pltpu.VMEM((NPCH, PCH), jnp.int32),
        pltpu.VMEM((NPCH, PCH), jnp.int32),
        pltpu.VMEM((NPCH, PCH), jnp.int32),
        pltpu.VMEM((NPCH, PCH), jnp.int32),
        pltpu.VMEM((2 * NPCH, PCH, 128), jnp.float32),
        pltpu.VMEM((sb,), jnp.int32),
        pltpu.VMEM((sb,), jnp.int32),
        pltpu.VMEM((sb, 128), jnp.float32),
        pltpu.VMEM((sb, 128), jnp.float32),
        pltpu.VMEM((20000,), jnp.int32),
        pltpu.VMEM((2 * KROWS,), jnp.int32),
        pltpu.VMEM((KBUF,), jnp.int32),
        pltpu.VMEM((512,), jnp.int32),
        pltpu.VMEM((16,), jnp.int32),
        pltpu.VMEM((16, 16), jnp.int32),
        pltpu.VMEM_SHARED((16, 16), jnp.int32),
        pltpu.SemaphoreType.DMA,
        pltpu.SemaphoreType.DMA,
        pltpu.SemaphoreType.DMA,
        pltpu.SemaphoreType.DMA,
        pltpu.SemaphoreType.DMA,
        pltpu.SemaphoreType.DMA,
    ]

    @functools.partial(pl.kernel, mesh=mesh, out_type=out_type,
                       scratch_types=scratch,
                       compiler_params=pltpu.CompilerParams(
                           needs_layout_passes=False))
    def k(f0_h, f1_h, mflat_h, off0_h, off1_h, s0_h, s1_h, invT_h,
          posf0_o, posf1_o, subf0_o, subf1_o, pi0_o, pi1_o,
          ck0_o, ck1_o, cnt_o,
          off0v, off1v, pidx0v, pidx1v, prows, sidx0v, sidx1v,
          srows0, srows1, invbuf, mbuf, kbuf, sentbuf, cstage, cntv,
          shared_cnt, sem_i, sem_m, sem_s, sem_f, sem_o, sem_k):
        c = lax.axis_index("c")
        s = lax.axis_index("s")
        wid = s * 2 + c
        pbase = wid * PB
        sbase = wid * sb

        h_inv = pltpu.async_copy(invT_h.at[c], invbuf, sem_k)
        mstart = 2 * jnp.maximum(
            jnp.minimum(s * KROWS, n_pairs - KROWS), 0)
        h_mb = pltpu.async_copy(
            mflat_h.at[pl.ds(pl.multiple_of(mstart, 8), 2 * KROWS)],
            mbuf, sem_k)

        h_idx = [
            pltpu.async_copy(off0_h.at[wid], off0v, sem_i),
            pltpu.async_copy(off1_h.at[wid], off1v, sem_i),
            pltpu.async_copy(s0_h.at[wid], sidx0v, sem_i),
            pltpu.async_copy(s1_h.at[wid], sidx1v, sem_i),
        ]
        for h in h_idx:
            h.wait()

        h_m = []
        for cc in range(NPCH):
            h_m.append(pltpu.async_copy(mflat_h.at[off0v.at[cc]],
                                        pidx0v.at[cc], sem_m))
            h_m.append(pltpu.async_copy(mflat_h.at[off1v.at[cc]],
                                        pidx1v.at[cc], sem_m))
        h_s0 = pltpu.async_copy(f0_h.at[sidx0v], srows0, sem_s)
        h_s1 = pltpu.async_copy(f1_h.at[sidx1v], srows1, sem_s)
        for h in h_m:
            h.wait()

        h_f = []
        for cc in range(NPCH):
            h_f.append(pltpu.async_copy(f0_h.at[pidx0v.at[cc]],
                                        prows.at[cc], sem_f))
            h_f.append(pltpu.async_copy(f1_h.at[pidx1v.at[cc]],
                                        prows.at[NPCH + cc], sem_f))
        h_o = [
            pltpu.async_copy(pidx0v, pi0_o.at[wid], sem_o),
            pltpu.async_copy(pidx1v, pi1_o.at[wid], sem_o),
        ]
        h_s0.wait()
        h_s1.wait()
        h_o.append(pltpu.async_copy(srows0, subf0_o.at[pl.ds(sbase, sb)], sem_o))
        h_o.append(pltpu.async_copy(srows1, subf1_o.at[pl.ds(sbase, sb)], sem_o))
        for h in h_f:
            h.wait()
        for cc in range(NPCH):
            h_o.append(pltpu.async_copy(
                prows.at[cc], posf0_o.at[pl.ds(pbase + cc * PCH, PCH)], sem_o))
            h_o.append(pltpu.async_copy(
                prows.at[NPCH + cc], posf1_o.at[pl.ds(pbase + cc * PCH, PCH)],
                sem_o))

        h_inv.wait()
        h_mb.wait()
        rowbase = mstart // 2
        lane = lax.iota(jnp.int32, 16)
        sent16 = jnp.full((16,), SENT, jnp.int32)
        cur = jnp.int32(0)
        for i in range(KROWS // 16):
            pos16 = (i * 16 + lane) * 2
            kv = plsc.load_gather(mbuf, [pos16 + c])
            rv = plsc.load_gather(mbuf, [pos16 + (1 - c)])
            rk = plsc.load_gather(invbuf, [rv])
            rowv = rowbase + i * 16 + lane
            mask = (rk >= 0) & (rowv < n_pairs)
            key = kv * 2048 + rk
            _, cv = plsc.sort_key_val(jnp.where(mask, 0, 1), key)
            pc = jnp.max(plsc.all_reduce_population_count(mask))
            cstage[...] = cv
            off = cur & 15
            cur_a = pl.multiple_of(cur & ~jnp.int32(15), 16)
            lpos = (lane - off) & 15
            rot = plsc.load_gather(cstage, [lpos])
            w1 = (lane >= off) & (lpos < pc)
            w2 = (lane < off) & (lpos < pc)
            v1 = kbuf[pl.ds(cur_a, 16)]
            kbuf[pl.ds(cur_a, 16)] = jnp.where(w1, rot, v1)
            v2 = kbuf[pl.ds(cur_a + 16, 16)]
            kbuf[pl.ds(cur_a + 16, 16)] = jnp.where(w2, rot, v2)
            cur = cur + pc
        off = cur & 15
        cur_a = pl.multiple_of(cur & ~jnp.int32(15), 16)
        vt = kbuf[pl.ds(cur_a, 16)]
        kbuf[pl.ds(cur_a, 16)] = jnp.where(lane >= off, sent16, vt)
        kbuf[pl.ds(cur_a + 16, 16)] = sent16
        rcnt = (cur + 7) & ~jnp.int32(7)

        cstage[...] = jnp.full((16,), rcnt, jnp.int32)
        pltpu.sync_copy(cstage, shared_cnt.at[s])
        plsc.subcore_barrier()
        pltpu.sync_copy(shared_cnt, cntv)
        rcnts = plsc.load_gather(cntv, [lane, jnp.zeros((16,), jnp.int32)])
        offset = jnp.sum(jnp.where(lane < s, rcnts, 0))
        total = jnp.sum(rcnts)

        def emit_copies(ck_o):
            for bit in (512, 256, 128, 64, 32, 16, 8):
                srcoff = rcnt & ~jnp.int32(2 * bit - 1)

                @pl.when((rcnt & bit) != 0)
                def _copy(bit=bit, srcoff=srcoff):
                    pltpu.sync_copy(
                        kbuf.at[pl.ds(pl.multiple_of(srcoff, 8), bit)],
                        ck_o.at[pl.ds(pl.multiple_of(offset + srcoff, 8),
                                      bit)])

            @pl.when(s == 0)
            def _tail():
                for j in range(32):
                    sentbuf[pl.ds(j * 16, 16)] = jnp.full((16,), SENT,
                                                          jnp.int32)
                pltpu.sync_copy(sentbuf,
                                ck_o.at[pl.ds(pl.multiple_of(total, 8), 512)])
                cstage[...] = jnp.full((16,), total, jnp.int32)
                pltpu.sync_copy(cstage, cnt_o.at[c])

        @pl.when(c == 0)
        def _side0():
            emit_copies(ck0_o)

        @pl.when(c == 1)
        def _side1():
            emit_copies(ck1_o)

        for h in h_o:
            h.wait()

    return k


def _tc_loss_kernel(posf0_ref, posf1_ref, subf0_ref, subf1_ref,
                    pi0_ref, pi1_ref, ck0_ref, ck1_ref, cnt_ref,
                    out_ref, acc_ref, b0sq_ref, b1sq_ref, m0_ref, m1_ref,
                    *, n_valid, n_sub, grid_m):
    p = pl.program_id(0)

    a0 = posf0_ref[...]
    a1 = posf1_ref[...]

    ones = jnp.ones((1, 128), jnp.float32)
    dotf = functools.partial(
        lax.dot_general,
        dimension_numbers=(((1,), (1,)), ((), ())),
        preferred_element_type=jnp.float32,
        precision=lax.Precision.HIGHEST,
    )
    dotb = functools.partial(
        lax.dot_general,
        dimension_numbers=(((1,), (1,)), ((), ())),
        preferred_element_type=jnp.float32,
    )

    @pl.when(p == 0)
    def _init():
        for i in range(5):
            acc_ref[i] = 0.0
        b0 = subf0_ref[...]
        b1 = subf1_ref[...]
        b0sq_ref[...] = dotf(ones, b0 * b0)
        b1sq_ref[...] = dotf(ones, b1 * b1)

    a0sq = jnp.sum(a0 * a0, axis=1, keepdims=True)
    a1sq = jnp.sum(a1 * a1, axis=1, keepdims=True)

    rows = p * BLK_M + lax.broadcasted_iota(jnp.int32, (BLK_M, 1), 0)
    valid = rows < n_valid
    jrow = lax.broadcasted_iota(jnp.int32, (BLK_M, n_sub), 1)

    def side(aq, asq, bsq, bmat_ref):
        g = dotb(aq.astype(jnp.bfloat16), bmat_ref[...].astype(jnp.bfloat16))
        d2 = jnp.maximum(asq + bsq - 2.0 * g, 0.0)
        bc = lax.bitcast_convert_type(d2, jnp.int32)
        key = jnp.bitwise_or(jnp.bitwise_and(bc, jnp.int32(~2047)), jrow)
        kmin = jnp.min(key, axis=1, keepdims=True)
        rank = jnp.bitwise_and(kmin, 2047)
        dmin = lax.bitcast_convert_type(
            jnp.bitwise_and(kmin, jnp.int32(~2047)), jnp.float32)
        dist = jnp.sqrt(dmin + 1e-07)
        nl = jnp.square(jnp.maximum(NEG_THRESH - dist, 0.0))
        return nl, rank

    nl0, rank0 = side(a0, a0sq, b1sq_ref[...], subf1_ref)
    nl1, rank1 = side(a1, a1sq, b0sq_ref[...], subf0_ref)

    q0 = pi0_ref[...] * 2048 + rank0
    q1 = pi1_ref[...] * 2048 + rank1
    cnt0 = cnt_ref[0, 0]
    cnt1 = cnt_ref[1, 0]

    m0_ref[...] = jnp.full((BLK_M, 1), SENT, jnp.int32)
    m1_ref[...] = jnp.full((BLK_M, 1), SENT, jnp.int32)
    for ci in range((CK_PAD - 512) // KCHUNK):
        @pl.when(ci * KCHUNK < cnt0)
        def _c0(ci=ci):
            ch = ck0_ref[:, pl.ds(ci * KCHUNK, KCHUNK)]
            x = jnp.min(jnp.bitwise_xor(q0, ch), axis=1, keepdims=True)
            m0_ref[...] = jnp.minimum(m0_ref[...], x)

        @pl.when(ci * KCHUNK < cnt1)
        def _c1(ci=ci):
            ch = ck1_ref[:, pl.ds(ci * KCHUNK, KCHUNK)]
            x = jnp.min(jnp.bitwise_xor(q1, ch), axis=1, keepdims=True)
            m1_ref[...] = jnp.minimum(m1_ref[...], x)

    mask0 = valid & (m0_ref[...] != 0)
    mask1 = valid & (m1_ref[...] != 0)

    dpos = a0 - a1
    pos_sq = jnp.sum(dpos * dpos, axis=1, keepdims=True)
    pos_term = jnp.where(valid, jnp.maximum(pos_sq - POS_THRESH, 0.0), 0.0)

    acc_ref[0] += jnp.sum(pos_term)
    acc_ref[1] += jnp.sum(jnp.where(mask0, nl0, 0.0))
    acc_ref[2] += jnp.sum(mask0.astype(jnp.float32))
    acc_ref[3] += jnp.sum(jnp.where(mask1, nl1, 0.0))
    acc_ref[4] += jnp.sum(mask1.astype(jnp.float32))

    @pl.when(p == grid_m - 1)
    def _fin():
        pos_loss = acc_ref[0] / n_valid
        neg0 = acc_ref[1] / jnp.maximum(acc_ref[2], 1.0)
        neg1 = acc_ref[3] / jnp.maximum(acc_ref[4], 1.0)
        out_ref[0, 0] = pos_loss + (neg0 + neg1) / 2.0


def _tc_loss(posF0, posF1, subF0, subF1, pi0, pi1, ck0, ck1, cnts, n_valid):
    n_sub = subF0.shape[0]
    grid_m = M_PAD // BLK_M
    kern = functools.partial(
        _tc_loss_kernel, n_valid=n_valid, n_sub=n_sub, grid_m=grid_m)
    full = lambda shape: pl.BlockSpec(shape, lambda p: (0, 0))
    out = pl.pallas_call(
        kern,
        grid=(grid_m,),
        in_specs=[
            pl.BlockSpec((BLK_M, 128), lambda p: (p, 0)),
            pl.BlockSpec((BLK_M, 128), lambda p: (p, 0)),
            full((n_sub, 128)),
            full((n_sub, 128)),
            pl.BlockSpec((BLK_M, 1), lambda p: (p, 0)),
            pl.BlockSpec((BLK_M, 1), lambda p: (p, 0)),
            full((1, CK_PAD)),
            full((1, CK_PAD)),
            pl.BlockSpec(memory_space=pltpu.SMEM),
        ],
        out_specs=pl.BlockSpec(memory_space=pltpu.SMEM),
        out_shape=jax.ShapeDtypeStruct((1, 1), jnp.float32),
        scratch_shapes=[
            pltpu.SMEM((8,), jnp.float32),
            pltpu.VMEM((1, n_sub), jnp.float32),
            pltpu.VMEM((1, n_sub), jnp.float32),
            pltpu.VMEM((BLK_M, 1), jnp.int32),
            pltpu.VMEM((BLK_M, 1), jnp.int32),
        ],
        compiler_params=pltpu.CompilerParams(
            dimension_semantics=("arbitrary",)),
    )(posF0, posF1, subF0, subF1, pi0, pi1, ck0, ck1, cnts)
    return out[0, 0]


def kernel(F0, F1, matches):
    N0, N1 = int(F0.shape[0]), int(F1.shape[0])
    n_pairs = int(matches.shape[0])
    sel0, sel1, pos_sel = _selections(N0, N1, n_pairs)
    n_valid = len(pos_sel)
    n_sub = len(sel0)

    pos_pad = np.zeros(M_PAD, np.int32)
    pos_pad[:n_valid] = pos_sel
    off0 = (2 * pos_pad).reshape(NW, NPCH, PCH)
    off1 = (2 * pos_pad + 1).reshape(NW, NPCH, PCH)
    s0w = sel0.reshape(NW, n_sub // NW)
    s1w = sel1.reshape(NW, n_sub // NW)
    invT = np.full((2, N0), -1, np.int32)
    invT[0, sel1] = np.arange(n_sub, dtype=np.int32)
    invT[1, sel0] = np.arange(n_sub, dtype=np.int32)

    matches = matches.astype(jnp.int32)
    mflat = matches.reshape(-1)

    (posF0, posF1, subF0, subF1, pi0, pi1,
     ck0, ck1, cnts) = _sc_gather_fn(n_sub, n_pairs)(
        F0, F1, mflat,
        jnp.asarray(off0), jnp.asarray(off1),
        jnp.asarray(s0w), jnp.asarray(s1w), jnp.asarray(invT))

    pi0 = pi0.reshape(M_PAD, 1)
    pi1 = pi1.reshape(M_PAD, 1)
    ck0 = ck0.reshape(1, CK_PAD)
    ck1 = ck1.reshape(1, CK_PAD)

    return _tc_loss(posF0, posF1, subF0, subF1, pi0, pi1, ck0, ck1,
                    cnts, n_valid)


# speedup vs baseline: 1.3420x; 1.3420x over previous
import functools

import numpy as np
import jax
import jax.numpy as jnp
from jax import lax
from jax.experimental import pallas as pl
from jax.experimental.pallas import tpu as pltpu
from jax.experimental.pallas import tpu_sc as plsc

POS_THRESH = 0.1
NEG_THRESH = 1.4
NUM_POS = 5192
NUM_HN_SAMPLES = 2048

NW = 32
NS = 16
PCH = 88
NPCH = 2
PB = PCH * NPCH
M_PAD = NW * PB
BLK_M = 352
KROWS = 640
KBUF = KROWS + 32
CK_PAD = NS * KROWS + 512
KCHUNK = 512
SENT = 0x7FFFFFFF


@functools.lru_cache(maxsize=None)
def _selections(N0, N1, n_pairs):
    rng = np.random.RandomState(0)
    sel0 = rng.choice(N0, min(N0, NUM_HN_SAMPLES), replace=False)
    sel1 = rng.choice(N1, min(N1, NUM_HN_SAMPLES), replace=False)
    if n_pairs > NUM_POS:
        pos_sel = rng.choice(n_pairs, NUM_POS, replace=False)
    else:
        pos_sel = np.arange(n_pairs)
    return sel0.astype(np.int32), sel1.astype(np.int32), pos_sel.astype(np.int32)


def _sc_gather_fn(n_sub, n_pairs):
    sb = n_sub // NW
    mesh = plsc.VectorSubcoreMesh(core_axis_name="c", subcore_axis_name="s")
    out_type = [
        jax.ShapeDtypeStruct((M_PAD, 128), jnp.float32),
        jax.ShapeDtypeStruct((M_PAD, 128), jnp.float32),
        jax.ShapeDtypeStruct((n_sub, 128), jnp.float32),
        jax.ShapeDtypeStruct((n_sub, 128), jnp.float32),
        jax.ShapeDtypeStruct((NW, NPCH, PCH), jnp.int32),
        jax.ShapeDtypeStruct((NW, NPCH, PCH), jnp.int32),
        jax.ShapeDtypeStruct((CK_PAD,), jnp.int32),
        jax.ShapeDtypeStruct((CK_PAD,), jnp.int32),
        jax.ShapeDtypeStruct((2, 16), jnp.int32),
    ]
    scratch = [
        pltpu.VMEM((NPCH, PCH), jnp.int32),
        pltpu.VMEM((NPCH, PCH), jnp.int32),
        pltpu.VMEM((NPCH, PCH), jnp.int32),
        pltpu.VMEM((NPCH, PCH), jnp.int32),
        pltpu.VMEM((2 * NPCH, PCH, 128), jnp.float32),
        pltpu.VMEM((sb,), jnp.int32),
        pltpu.VMEM((sb,), jnp.int32),
        pltpu.VMEM((sb, 128), jnp.float32),
        pltpu.VMEM((sb, 128), jnp.float32),
        pltpu.VMEM((20000,), jnp.int32),
        pltpu.VMEM((2 * KROWS,), jnp.int32),
        pltpu.VMEM((KBUF,), jnp.int32),
        pltpu.VMEM((512,), jnp.int32),
        pltpu.VMEM((16,), jnp.int32),
        pltpu.VMEM((16, 16), jnp.int32),
        pltpu.VMEM_SHARED((16, 16), jnp.int32),
        pltpu.SemaphoreType.DMA,
        pltpu.SemaphoreType.DMA,
        pltpu.SemaphoreType.DMA,
        pltpu.SemaphoreType.DMA,
        pltpu.SemaphoreType.DMA,
        pltpu.SemaphoreType.DMA,
    ]

    @functools.partial(pl.kernel, mesh=mesh, out_type=out_type,
                       scratch_types=scratch,
                       compiler_params=pltpu.CompilerParams(
                           needs_layout_passes=False))
    def k(f0_h, f1_h, mflat_h, off0_h, off1_h, s0_h, s1_h, invT_h,
          posf0_o, posf1_o, subf0_o, subf1_o, pi0_o, pi1_o,
          ck0_o, ck1_o, cnt_o,
          off0v, off1v, pidx0v, pidx1v, prows, sidx0v, sidx1v,
          srows0, srows1, invbuf, mbuf, kbuf, sentbuf, cstage, cntv,
          shared_cnt, sem_i, sem_m, sem_s, sem_f, sem_o, sem_k):
        c = lax.axis_index("c")
        s = lax.axis_index("s")
        wid = s * 2 + c
        pbase = wid * PB
        sbase = wid * sb

        h_inv = pltpu.async_copy(invT_h.at[c], invbuf, sem_k)
        mstart = 2 * jnp.maximum(
            jnp.minimum(s * KROWS, n_pairs - KROWS), 0)
        h_mb = pltpu.async_copy(
            mflat_h.at[pl.ds(pl.multiple_of(mstart, 8), 2 * KROWS)],
            mbuf, sem_k)

        h_idx = [
            pltpu.async_copy(off0_h.at[wid], off0v, sem_i),
            pltpu.async_copy(off1_h.at[wid], off1v, sem_i),
            pltpu.async_copy(s0_h.at[wid], sidx0v, sem_i),
            pltpu.async_copy(s1_h.at[wid], sidx1v, sem_i),
        ]
        for h in h_idx:
            h.wait()

        h_m = []
        for cc in range(NPCH):
            h_m.append(pltpu.async_copy(mflat_h.at[off0v.at[cc]],
                                        pidx0v.at[cc], sem_m))
            h_m.append(pltpu.async_copy(mflat_h.at[off1v.at[cc]],
                                        pidx1v.at[cc], sem_m))
        h_s0 = pltpu.async_copy(f0_h.at[sidx0v], srows0, sem_s)
        h_s1 = pltpu.async_copy(f1_h.at[sidx1v], srows1, sem_s)
        for h in h_m:
            h.wait()

        h_f = []
        for cc in range(NPCH):
            h_f.append(pltpu.async_copy(f0_h.at[pidx0v.at[cc]],
                                        prows.at[cc], sem_f))
            h_f.append(pltpu.async_copy(f1_h.at[pidx1v.at[cc]],
                                        prows.at[NPCH + cc], sem_f))
        h_o = [
            pltpu.async_copy(pidx0v, pi0_o.at[wid], sem_o),
            pltpu.async_copy(pidx1v, pi1_o.at[wid], sem_o),
        ]
        h_s0.wait()
        h_s1.wait()
        h_o.append(pltpu.async_copy(srows0, subf0_o.at[pl.ds(sbase, sb)], sem_o))
        h_o.append(pltpu.async_copy(srows1, subf1_o.at[pl.ds(sbase, sb)], sem_o))
        for h in h_f:
            h.wait()
        for cc in range(NPCH):
            h_o.append(pltpu.async_copy(
                prows.at[cc], posf0_o.at[pl.ds(pbase + cc * PCH, PCH)], sem_o))
            h_o.append(pltpu.async_copy(
                prows.at[NPCH + cc], posf1_o.at[pl.ds(pbase + cc * PCH, PCH)],
                sem_o))

        h_inv.wait()
        h_mb.wait()
        rowbase = mstart // 2
        lane = lax.iota(jnp.int32, 16)
        sent16 = jnp.full((16,), SENT, jnp.int32)
        cur = jnp.int32(0)
        for i in range(KROWS // 16):
            pos16 = (i * 16 + lane) * 2
            kv = plsc.load_gather(mbuf, [pos16 + c])
            rv = plsc.load_gather(mbuf, [pos16 + (1 - c)])
            rk = plsc.load_gather(invbuf, [rv])
            rowv = rowbase + i * 16 + lane
            mask = (rk >= 0) & (rowv < n_pairs)
            key = kv * 2048 + rk
            _, cv = plsc.sort_key_val(jnp.where(mask, 0, 1), key)
            pc = jnp.max(plsc.all_reduce_population_count(mask))
            cstage[...] = cv
            off = cur & 15
            cur_a = pl.multiple_of(cur & ~jnp.int32(15), 16)
            lpos = (lane - off) & 15
            rot = plsc.load_gather(cstage, [lpos])
            w1 = (lane >= off) & (lpos < pc)
            w2 = (lane < off) & (lpos < pc)
            v1 = kbuf[pl.ds(cur_a, 16)]
            kbuf[pl.ds(cur_a, 16)] = jnp.where(w1, rot, v1)
            v2 = kbuf[pl.ds(cur_a + 16, 16)]
            kbuf[pl.ds(cur_a + 16, 16)] = jnp.where(w2, rot, v2)
            cur = cur + pc
        off = cur & 15
        cur_a = pl.multiple_of(cur & ~jnp.int32(15), 16)
        vt = kbuf[pl.ds(cur_a, 16)]
        kbuf[pl.ds(cur_a, 16)] = jnp.where(lane >= off, sent16, vt)
        kbuf[pl.ds(cur_a + 16, 16)] = sent16
        rcnt = (cur + 7) & ~jnp.int32(7)

        cstage[...] = jnp.full((16,), rcnt, jnp.int32)
        pltpu.sync_copy(cstage, shared_cnt.at[s])
        plsc.subcore_barrier()
        pltpu.sync_copy(shared_cnt, cntv)
        rcnts = plsc.load_gather(cntv, [lane, jnp.zeros((16,), jnp.int32)])
        offset = jnp.sum(jnp.where(lane < s, rcnts, 0))
        total = jnp.sum(rcnts)

        def emit_copies(ck_o):
            for bit in (512, 256, 128, 64, 32, 16, 8):
                srcoff = rcnt & ~jnp.int32(2 * bit - 1)

                @pl.when((rcnt & bit) != 0)
                def _copy(bit=bit, srcoff=srcoff):
                    pltpu.sync_copy(
                        kbuf.at[pl.ds(pl.multiple_of(srcoff, 8), bit)],
                        ck_o.at[pl.ds(pl.multiple_of(offset + srcoff, 8),
                                      bit)])

            @pl.when(s == 0)
            def _tail():
                for j in range(32):
                    sentbuf[pl.ds(j * 16, 16)] = jnp.full((16,), SENT,
                                                          jnp.int32)
                pltpu.sync_copy(sentbuf,
                                ck_o.at[pl.ds(pl.multiple_of(total, 8), 512)])
                cstage[...] = jnp.full((16,), total, jnp.int32)
                pltpu.sync_copy(cstage, cnt_o.at[c])

        @pl.when(c == 0)
        def _side0():
            emit_copies(ck0_o)

        @pl.when(c == 1)
        def _side1():
            emit_copies(ck1_o)

        for h in h_o:
            h.wait()

    return k


def _tc_loss_kernel(posf0_ref, posf1_ref, subf0_ref, subf1_ref,
                    pi0_ref, pi1_ref, ck0_ref, ck1_ref, cnt_ref,
                    out_ref, acc_ref, b0sq_ref, b1sq_ref, m0_ref, m1_ref,
                    *, n_valid, n_sub, grid_m):
    p = pl.program_id(0)

    a0 = posf0_ref[...]
    a1 = posf1_ref[...]

    ones = jnp.ones((1, 128), jnp.float32)
    dotf = functools.partial(
        lax.dot_general,
        dimension_numbers=(((1,), (1,)), ((), ())),
        preferred_element_type=jnp.float32,
        precision=lax.Precision.HIGHEST,
    )
    dotb = functools.partial(
        lax.dot_general,
        dimension_numbers=(((1,), (1,)), ((), ())),
        preferred_element_type=jnp.float32,
    )

    @pl.when(p == 0)
    def _init():
        for i in range(5):
            acc_ref[i] = 0.0
        b0 = subf0_ref[...]
        b1 = subf1_ref[...]
        b0sq_ref[...] = dotf(ones, b0 * b0)
        b1sq_ref[...] = dotf(ones, b1 * b1)

    a0sq = jnp.sum(a0 * a0, axis=1, keepdims=True)
    a1sq = jnp.sum(a1 * a1, axis=1, keepdims=True)

    rows = p * BLK_M + lax.broadcasted_iota(jnp.int32, (BLK_M, 1), 0)
    valid = rows < n_valid
    jrow = lax.broadcasted_iota(jnp.int32, (BLK_M, n_sub), 1)

    def side(aq, asq, bsq, bmat_ref):
        g = dotb(aq.astype(jnp.bfloat16), bmat_ref[...].astype(jnp.bfloat16))
        d2 = jnp.maximum(asq + bsq - 2.0 * g, 0.0)
        bc = lax.bitcast_convert_type(d2, jnp.int32)
        key = jnp.bitwise_or(jnp.bitwise_and(bc, jnp.int32(~2047)), jrow)
        kmin = jnp.min(key, axis=1, keepdims=True)
        rank = jnp.bitwise_and(kmin, 2047)
        dmin = lax.bitcast_convert_type(
            jnp.bitwise_and(kmin, jnp.int32(~2047)), jnp.float32)
        dist = jnp.sqrt(dmin + 1e-07)
        nl = jnp.square(jnp.maximum(NEG_THRESH - dist, 0.0))
        return nl, rank

    nl0, rank0 = side(a0, a0sq, b1sq_ref[...], subf1_ref)
    nl1, rank1 = side(a1, a1sq, b0sq_ref[...], subf0_ref)

    q0 = pi0_ref[...] * 2048 + rank0
    q1 = pi1_ref[...] * 2048 + rank1
    cnt0 = cnt_ref[0, 0]
    cnt1 = cnt_ref[1, 0]

    m0_ref[...] = jnp.full((BLK_M, 1), SENT, jnp.int32)
    m1_ref[...] = jnp.full((BLK_M, 1), SENT, jnp.int32)
    for ci in range((CK_PAD - 512) // KCHUNK):
        @pl.when(ci * KCHUNK < cnt0)
        def _c0(ci=ci):
            ch = ck0_ref[:, pl.ds(ci * KCHUNK, KCHUNK)]
            x = jnp.min(jnp.bitwise_xor(q0, ch), axis=1, keepdims=True)
            m0_ref[...] = jnp.minimum(m0_ref[...], x)

        @pl.when(ci * KCHUNK < cnt1)
        def _c1(ci=ci):
            ch = ck1_ref[:, pl.ds(ci * KCHUNK, KCHUNK)]
            x = jnp.min(jnp.bitwise_xor(q1, ch), axis=1, keepdims=True)
            m1_ref[...] = jnp.minimum(m1_ref[...], x)

    mask0 = valid & (m0_ref[...] != 0)
    mask1 = valid & (m1_ref[...] != 0)

    dpos = a0 - a1
    pos_sq = jnp.sum(dpos * dpos, axis=1, keepdims=True)
    pos_term = jnp.where(valid, jnp.maximum(pos_sq - POS_THRESH, 0.0), 0.0)

    acc_ref[0] += jnp.sum(pos_term)
    acc_ref[1] += jnp.sum(jnp.where(mask0, nl0, 0.0))
    acc_ref[2] += jnp.sum(mask0.astype(jnp.float32))
    acc_ref[3] += jnp.sum(jnp.where(mask1, nl1, 0.0))
    acc_ref[4] += jnp.sum(mask1.astype(jnp.float32))

    @pl.when(p == grid_m - 1)
    def _fin():
        pos_loss = acc_ref[0] / n_valid
        neg0 = acc_ref[1] / jnp.maximum(acc_ref[2], 1.0)
        neg1 = acc_ref[3] / jnp.maximum(acc_ref[4], 1.0)
        out_ref[0, 0] = pos_loss + (neg0 + neg1) / 2.0


def _tc_loss(posF0, posF1, subF0, subF1, pi0, pi1, ck0, ck1, cnts, n_valid):
    n_sub = subF0.shape[0]
    grid_m = M_PAD // BLK_M
    kern = functools.partial(
        _tc_loss_kernel, n_valid=n_valid, n_sub=n_sub, grid_m=grid_m)
    full = lambda shape: pl.BlockSpec(shape, lambda p: (0, 0))
    out = pl.pallas_call(
        kern,
        grid=(grid_m,),
        in_specs=[
            pl.BlockSpec((BLK_M, 128), lambda p: (p, 0)),
            pl.BlockSpec((BLK_M, 128), lambda p: (p, 0)),
            full((n_sub, 128)),
            full((n_sub, 128)),
            pl.BlockSpec((BLK_M, 1), lambda p: (p, 0)),
            pl.BlockSpec((BLK_M, 1), lambda p: (p, 0)),
            full((1, CK_PAD)),
            full((1, CK_PAD)),
            pl.BlockSpec(memory_space=pltpu.SMEM),
        ],
        out_specs=pl.BlockSpec(memory_space=pltpu.SMEM),
        out_shape=jax.ShapeDtypeStruct((1, 1), jnp.float32),
        scratch_shapes=[
            pltpu.SMEM((8,), jnp.float32),
            pltpu.VMEM((1, n_sub), jnp.float32),
            pltpu.VMEM((1, n_sub), jnp.float32),
            pltpu.VMEM((BLK_M, 1), jnp.int32),
            pltpu.VMEM((BLK_M, 1), jnp.int32),
        ],
        compiler_params=pltpu.CompilerParams(
            dimension_semantics=("arbitrary",)),
    )(posF0, posF1, subF0, subF1, pi0, pi1, ck0, ck1, cnts)
    return out[0, 0]


def kernel(F0, F1, matches):
    N0, N1 = int(F0.shape[0]), int(F1.shape[0])
    n_pairs = int(matches.shape[0])
    sel0, sel1, pos_sel = _selections(N0, N1, n_pairs)
    n_valid = len(pos_sel)
    n_sub = len(sel0)

    pos_pad = np.zeros(M_PAD, np.int32)
    pos_pad[:n_valid] = pos_sel
    off0 = (2 * pos_pad).reshape(NW, NPCH, PCH)
    off1 = (2 * pos_pad + 1).reshape(NW, NPCH, PCH)
    s0w = sel0.reshape(NW, n_sub // NW)
    s1w = sel1.reshape(NW, n_sub // NW)
    invT = np.full((2, N0), -1, np.int32)
    invT[0, sel1] = np.arange(n_sub, dtype=np.int32)
    invT[1, sel0] = np.arange(n_sub, dtype=np.int32)

    matches = matches.astype(jnp.int32)
    mflat = matches.reshape(-1)

    (posF0, posF1, subF0, subF1, pi0, pi1,
     ck0, ck1, cnts) = _sc_gather_fn(n_sub, n_pairs)(
        F0, F1, mflat,
        jnp.asarray(off0), jnp.asarray(off1),
        jnp.asarray(s0w), jnp.asarray(s1w), jnp.asarray(invT))

    pi0 = pi0.reshape(M_PAD, 1)
    pi1 = pi1.reshape(M_PAD, 1)
    ck0 = ck0.reshape(1, CK_PAD)
    ck1 = ck1.reshape(1, CK_PAD)

    return _tc_loss(posF0, posF1, subF0, subF1, pi0, pi1, ck0, ck1,
                    cnts, n_valid)


# speedup vs baseline: 1.5641x; 1.1656x over previous
import functools

import numpy as np
import jax
import jax.numpy as jnp
from jax import lax
from jax.experimental import pallas as pl
from jax.experimental.pallas import tpu as pltpu
from jax.experimental.pallas import tpu_sc as plsc

POS_THRESH = 0.1
NEG_THRESH = 1.4
NUM_POS = 5192
NUM_HN_SAMPLES = 2048

NW = 32
NS = 16
PCH = 88
NPCH = 2
PB = PCH * NPCH
M_PAD = NW * PB
BLK_M = 704
KROWS = 640
KBUF = KROWS + 32
CK_PAD = NS * KROWS + 512
KCHUNK = 512
SENT = 0x7FFFFFFF


@functools.lru_cache(maxsize=None)
def _selections(N0, N1, n_pairs):
    rng = np.random.RandomState(0)
    sel0 = rng.choice(N0, min(N0, NUM_HN_SAMPLES), replace=False)
    sel1 = rng.choice(N1, min(N1, NUM_HN_SAMPLES), replace=False)
    if n_pairs > NUM_POS:
        pos_sel = rng.choice(n_pairs, NUM_POS, replace=False)
    else:
        pos_sel = np.arange(n_pairs)
    return sel0.astype(np.int32), sel1.astype(np.int32), pos_sel.astype(np.int32)


def _sc_gather_fn(n_sub, n_pairs):
    sb = n_sub // NW
    mesh = plsc.VectorSubcoreMesh(core_axis_name="c", subcore_axis_name="s")
    out_type = [
        jax.ShapeDtypeStruct((M_PAD, 128), jnp.float32),
        jax.ShapeDtypeStruct((M_PAD, 128), jnp.float32),
        jax.ShapeDtypeStruct((n_sub, 128), jnp.float32),
        jax.ShapeDtypeStruct((n_sub, 128), jnp.float32),
        jax.ShapeDtypeStruct((NW, NPCH, PCH), jnp.int32),
        jax.ShapeDtypeStruct((NW, NPCH, PCH), jnp.int32),
        jax.ShapeDtypeStruct((CK_PAD,), jnp.int32),
        jax.ShapeDtypeStruct((CK_PAD,), jnp.int32),
        jax.ShapeDtypeStruct((2, 16), jnp.int32),
    ]
    scratch = [
        pltpu.VMEM((NPCH, PCH), jnp.int32),
        pltpu.VMEM((NPCH, PCH), jnp.int32),
        pltpu.VMEM((NPCH, PCH), jnp.int32),
        pltpu.VMEM((NPCH, PCH), jnp.int32),
        pltpu.VMEM((2 * NPCH, PCH, 128), jnp.float32),
        pltpu.VMEM((sb,), jnp.int32),
        pltpu.VMEM((sb,), jnp.int32),
        pltpu.VMEM((sb, 128), jnp.float32),
        pltpu.VMEM((sb, 128), jnp.float32),
        pltpu.VMEM((20000,), jnp.int32),
        pltpu.VMEM((2 * KROWS,), jnp.int32),
        pltpu.VMEM((KBUF,), jnp.int32),
        pltpu.VMEM((512,), jnp.int32),
        pltpu.VMEM((16,), jnp.int32),
        pltpu.VMEM((16, 16), jnp.int32),
        pltpu.VMEM_SHARED((16, 16), jnp.int32),
        pltpu.SemaphoreType.DMA,
        pltpu.SemaphoreType.DMA,
        pltpu.SemaphoreType.DMA,
        pltpu.SemaphoreType.DMA,
        pltpu.SemaphoreType.DMA,
        pltpu.SemaphoreType.DMA,
    ]

    @functools.partial(pl.kernel, mesh=mesh, out_type=out_type,
                       scratch_types=scratch,
                       compiler_params=pltpu.CompilerParams(
                           needs_layout_passes=False))
    def k(f0_h, f1_h, mflat_h, off0_h, off1_h, s0_h, s1_h, invT_h,
          posf0_o, posf1_o, subf0_o, subf1_o, pi0_o, pi1_o,
          ck0_o, ck1_o, cnt_o,
          off0v, off1v, pidx0v, pidx1v, prows, sidx0v, sidx1v,
          srows0, srows1, invbuf, mbuf, kbuf, sentbuf, cstage, cntv,
          shared_cnt, sem_i, sem_m, sem_s, sem_f, sem_o, sem_k):
        c = lax.axis_index("c")
        s = lax.axis_index("s")
        wid = s * 2 + c
        pbase = wid * PB
        sbase = wid * sb

        h_inv = pltpu.async_copy(invT_h.at[c], invbuf, sem_k)
        mstart = 2 * jnp.maximum(
            jnp.minimum(s * KROWS, n_pairs - KROWS), 0)
        h_mb = pltpu.async_copy(
            mflat_h.at[pl.ds(pl.multiple_of(mstart, 8), 2 * KROWS)],
            mbuf, sem_k)

        h_idx = [
            pltpu.async_copy(off0_h.at[wid], off0v, sem_i),
            pltpu.async_copy(off1_h.at[wid], off1v, sem_i),
            pltpu.async_copy(s0_h.at[wid], sidx0v, sem_i),
            pltpu.async_copy(s1_h.at[wid], sidx1v, sem_i),
        ]
        for h in h_idx:
            h.wait()

        h_m = []
        for cc in range(NPCH):
            h_m.append(pltpu.async_copy(mflat_h.at[off0v.at[cc]],
                                        pidx0v.at[cc], sem_m))
            h_m.append(pltpu.async_copy(mflat_h.at[off1v.at[cc]],
                                        pidx1v.at[cc], sem_m))
        h_s0 = pltpu.async_copy(f0_h.at[sidx0v], srows0, sem_s)
        h_s1 = pltpu.async_copy(f1_h.at[sidx1v], srows1, sem_s)
        for h in h_m:
            h.wait()

        h_f = []
        for cc in range(NPCH):
            h_f.append(pltpu.async_copy(f0_h.at[pidx0v.at[cc]],
                                        prows.at[cc], sem_f))
            h_f.append(pltpu.async_copy(f1_h.at[pidx1v.at[cc]],
                                        prows.at[NPCH + cc], sem_f))
        h_o = [
            pltpu.async_copy(pidx0v, pi0_o.at[wid], sem_o),
            pltpu.async_copy(pidx1v, pi1_o.at[wid], sem_o),
        ]
        h_s0.wait()
        h_s1.wait()
        h_o.append(pltpu.async_copy(srows0, subf0_o.at[pl.ds(sbase, sb)], sem_o))
        h_o.append(pltpu.async_copy(srows1, subf1_o.at[pl.ds(sbase, sb)], sem_o))
        for h in h_f:
            h.wait()
        for cc in range(NPCH):
            h_o.append(pltpu.async_copy(
                prows.at[cc], posf0_o.at[pl.ds(pbase + cc * PCH, PCH)], sem_o))
            h_o.append(pltpu.async_copy(
                prows.at[NPCH + cc], posf1_o.at[pl.ds(pbase + cc * PCH, PCH)],
                sem_o))

        h_inv.wait()
        h_mb.wait()
        rowbase = mstart // 2
        lane = lax.iota(jnp.int32, 16)
        sent16 = jnp.full((16,), SENT, jnp.int32)
        cur = jnp.int32(0)
        for i in range(KROWS // 16):
            pos16 = (i * 16 + lane) * 2
            kv = plsc.load_gather(mbuf, [pos16 + c])
            rv = plsc.load_gather(mbuf, [pos16 + (1 - c)])
            rk = plsc.load_gather(invbuf, [rv])
            rowv = rowbase + i * 16 + lane
            mask = (rk >= 0) & (rowv < n_pairs)
            key = kv * 2048 + rk
            _, cv = plsc.sort_key_val(jnp.where(mask, 0, 1), key)
            pc = jnp.max(plsc.all_reduce_population_count(mask))
            cstage[...] = cv
            off = cur & 15
            cur_a = pl.multiple_of(cur & ~jnp.int32(15), 16)
            lpos = (lane - off) & 15
            rot = plsc.load_gather(cstage, [lpos])
            w1 = (lane >= off) & (lpos < pc)
            w2 = (lane < off) & (lpos < pc)
            v1 = kbuf[pl.ds(cur_a, 16)]
            kbuf[pl.ds(cur_a, 16)] = jnp.where(w1, rot, v1)
            v2 = kbuf[pl.ds(cur_a + 16, 16)]
            kbuf[pl.ds(cur_a + 16, 16)] = jnp.where(w2, rot, v2)
            cur = cur + pc
        off = cur & 15
        cur_a = pl.multiple_of(cur & ~jnp.int32(15), 16)
        vt = kbuf[pl.ds(cur_a, 16)]
        kbuf[pl.ds(cur_a, 16)] = jnp.where(lane >= off, sent16, vt)
        kbuf[pl.ds(cur_a + 16, 16)] = sent16
        rcnt = (cur + 7) & ~jnp.int32(7)

        cstage[...] = jnp.full((16,), rcnt, jnp.int32)
        pltpu.sync_copy(cstage, shared_cnt.at[s])
        plsc.subcore_barrier()
        pltpu.sync_copy(shared_cnt, cntv)
        rcnts = plsc.load_gather(cntv, [lane, jnp.zeros((16,), jnp.int32)])
        offset = jnp.sum(jnp.where(lane < s, rcnts, 0))
        total = jnp.sum(rcnts)

        def emit_copies(ck_o):
            for bit in (512, 256, 128, 64, 32, 16, 8):
                srcoff = rcnt & ~jnp.int32(2 * bit - 1)

                @pl.when((rcnt & bit) != 0)
                def _copy(bit=bit, srcoff=srcoff):
                    pltpu.sync_copy(
                        kbuf.at[pl.ds(pl.multiple_of(srcoff, 8), bit)],
                        ck_o.at[pl.ds(pl.multiple_of(offset + srcoff, 8),
                                      bit)])

            @pl.when(s == 0)
            def _tail():
                for j in range(32):
                    sentbuf[pl.ds(j * 16, 16)] = jnp.full((16,), SENT,
                                                          jnp.int32)
                pltpu.sync_copy(sentbuf,
                                ck_o.at[pl.ds(pl.multiple_of(total, 8), 512)])
                cstage[...] = jnp.full((16,), total, jnp.int32)
                pltpu.sync_copy(cstage, cnt_o.at[c])

        @pl.when(c == 0)
        def _side0():
            emit_copies(ck0_o)

        @pl.when(c == 1)
        def _side1():
            emit_copies(ck1_o)

        for h in h_o:
            h.wait()

    return k


def _tc_loss_kernel(posf0_ref, posf1_ref, subf0_ref, subf1_ref,
                    pi0_ref, pi1_ref, ck0_ref, ck1_ref, cnt_ref,
                    out_ref, acc_ref, ba0_ref, ba1_ref, m0_ref, m1_ref,
                    *, n_valid, n_sub, grid_m):
    p = pl.program_id(0)

    a0 = posf0_ref[...]
    a1 = posf1_ref[...]

    dotb = functools.partial(
        lax.dot_general,
        dimension_numbers=(((1,), (1,)), ((), ())),
        preferred_element_type=jnp.float32,
    )

    def augment(x, col1, col2):
        return jnp.concatenate([x, col1, col2], axis=1).astype(jnp.bfloat16)

    @pl.when(p == 0)
    def _init():
        for i in range(5):
            acc_ref[i] = 0.0
        b0 = subf0_ref[...]
        b1 = subf1_ref[...]
        onesb = jnp.ones((n_sub, 1), jnp.float32)
        ba0_ref[...] = augment(-2.0 * b0,
                               jnp.sum(b0 * b0, axis=1, keepdims=True), onesb)
        ba1_ref[...] = augment(-2.0 * b1,
                               jnp.sum(b1 * b1, axis=1, keepdims=True), onesb)

    a0sq = jnp.sum(a0 * a0, axis=1, keepdims=True)
    a1sq = jnp.sum(a1 * a1, axis=1, keepdims=True)
    onesa = jnp.ones((BLK_M, 1), jnp.float32)
    aa0 = augment(a0, onesa, a0sq)
    aa1 = augment(a1, onesa, a1sq)

    rows = p * BLK_M + lax.broadcasted_iota(jnp.int32, (BLK_M, 1), 0)
    valid = rows < n_valid
    jrow = lax.broadcasted_iota(jnp.int32, (BLK_M, n_sub), 1)

    def side(aaug, ba_ref):
        d2 = jnp.maximum(dotb(aaug, ba_ref[...]), 0.0)
        bc = lax.bitcast_convert_type(d2, jnp.int32)
        key = jnp.bitwise_or(jnp.bitwise_and(bc, jnp.int32(~2047)), jrow)
        kmin = jnp.min(key, axis=1, keepdims=True)
        rank = jnp.bitwise_and(kmin, 2047)
        dmin = lax.bitcast_convert_type(
            jnp.bitwise_and(kmin, jnp.int32(~2047)), jnp.float32)
        dist = jnp.sqrt(dmin + 1e-07)
        nl = jnp.square(jnp.maximum(NEG_THRESH - dist, 0.0))
        return nl, rank

    nl0, rank0 = side(aa0, ba1_ref)
    nl1, rank1 = side(aa1, ba0_ref)

    q0 = pi0_ref[...] * 2048 + rank0
    q1 = pi1_ref[...] * 2048 + rank1
    cnt0 = cnt_ref[0, 0]
    cnt1 = cnt_ref[1, 0]

    m0_ref[...] = jnp.full((BLK_M, 1), SENT, jnp.int32)
    m1_ref[...] = jnp.full((BLK_M, 1), SENT, jnp.int32)
    for ci in range((CK_PAD - 512) // KCHUNK):
        @pl.when(ci * KCHUNK < cnt0)
        def _c0(ci=ci):
            ch = ck0_ref[:, pl.ds(ci * KCHUNK, KCHUNK)]
            x = jnp.min(jnp.bitwise_xor(q0, ch), axis=1, keepdims=True)
            m0_ref[...] = jnp.minimum(m0_ref[...], x)

        @pl.when(ci * KCHUNK < cnt1)
        def _c1(ci=ci):
            ch = ck1_ref[:, pl.ds(ci * KCHUNK, KCHUNK)]
            x = jnp.min(jnp.bitwise_xor(q1, ch), axis=1, keepdims=True)
            m1_ref[...] = jnp.minimum(m1_ref[...], x)

    mask0 = valid & (m0_ref[...] != 0)
    mask1 = valid & (m1_ref[...] != 0)

    dpos = a0 - a1
    pos_sq = jnp.sum(dpos * dpos, axis=1, keepdims=True)
    pos_term = jnp.where(valid, jnp.maximum(pos_sq - POS_THRESH, 0.0), 0.0)

    acc_ref[0] += jnp.sum(pos_term)
    acc_ref[1] += jnp.sum(jnp.where(mask0, nl0, 0.0))
    acc_ref[2] += jnp.sum(mask0.astype(jnp.float32))
    acc_ref[3] += jnp.sum(jnp.where(mask1, nl1, 0.0))
    acc_ref[4] += jnp.sum(mask1.astype(jnp.float32))

    @pl.when(p == grid_m - 1)
    def _fin():
        pos_loss = acc_ref[0] / n_valid
        neg0 = acc_ref[1] / jnp.maximum(acc_ref[2], 1.0)
        neg1 = acc_ref[3] / jnp.maximum(acc_ref[4], 1.0)
        out_ref[0, 0] = pos_loss + (neg0 + neg1) / 2.0


def _tc_loss(posF0, posF1, subF0, subF1, pi0, pi1, ck0, ck1, cnts, n_valid):
    n_sub = subF0.shape[0]
    grid_m = M_PAD // BLK_M
    kern = functools.partial(
        _tc_loss_kernel, n_valid=n_valid, n_sub=n_sub, grid_m=grid_m)
    full = lambda shape: pl.BlockSpec(shape, lambda p: (0, 0))
    out = pl.pallas_call(
        kern,
        grid=(grid_m,),
        in_specs=[
            pl.BlockSpec((BLK_M, 128), lambda p: (p, 0)),
            pl.BlockSpec((BLK_M, 128), lambda p: (p, 0)),
            full((n_sub, 128)),
            full((n_sub, 128)),
            pl.BlockSpec((BLK_M, 1), lambda p: (p, 0)),
            pl.BlockSpec((BLK_M, 1), lambda p: (p, 0)),
            full((1, CK_PAD)),
            full((1, CK_PAD)),
            pl.BlockSpec(memory_space=pltpu.SMEM),
        ],
        out_specs=pl.BlockSpec(memory_space=pltpu.SMEM),
        out_shape=jax.ShapeDtypeStruct((1, 1), jnp.float32),
        scratch_shapes=[
            pltpu.SMEM((8,), jnp.float32),
            pltpu.VMEM((n_sub, 130), jnp.bfloat16),
            pltpu.VMEM((n_sub, 130), jnp.bfloat16),
            pltpu.VMEM((BLK_M, 1), jnp.int32),
            pltpu.VMEM((BLK_M, 1), jnp.int32),
        ],
        compiler_params=pltpu.CompilerParams(
            dimension_semantics=("arbitrary",)),
    )(posF0, posF1, subF0, subF1, pi0, pi1, ck0, ck1, cnts)
    return out[0, 0]


def kernel(F0, F1, matches):
    N0, N1 = int(F0.shape[0]), int(F1.shape[0])
    n_pairs = int(matches.shape[0])
    sel0, sel1, pos_sel = _selections(N0, N1, n_pairs)
    n_valid = len(pos_sel)
    n_sub = len(sel0)

    pos_pad = np.zeros(M_PAD, np.int32)
    pos_pad[:n_valid] = pos_sel
    off0 = (2 * pos_pad).reshape(NW, NPCH, PCH)
    off1 = (2 * pos_pad + 1).reshape(NW, NPCH, PCH)
    s0w = sel0.reshape(NW, n_sub // NW)
    s1w = sel1.reshape(NW, n_sub // NW)
    invT = np.full((2, N0), -1, np.int32)
    invT[0, sel1] = np.arange(n_sub, dtype=np.int32)
    invT[1, sel0] = np.arange(n_sub, dtype=np.int32)

    matches = matches.astype(jnp.int32)
    mflat = matches.reshape(-1)

    (posF0, posF1, subF0, subF1, pi0, pi1,
     ck0, ck1, cnts) = _sc_gather_fn(n_sub, n_pairs)(
        F0, F1, mflat,
        jnp.asarray(off0), jnp.asarray(off1),
        jnp.asarray(s0w), jnp.asarray(s1w), jnp.asarray(invT))

    pi0 = pi0.reshape(M_PAD, 1)
    pi1 = pi1.reshape(M_PAD, 1)
    ck0 = ck0.reshape(1, CK_PAD)
    ck1 = ck1.reshape(1, CK_PAD)

    return _tc_loss(posF0, posF1, subF0, subF1, pi0, pi1, ck0, ck1,
                    cnts, n_valid)


# speedup vs baseline: 1.5656x; 1.0009x over previous
import functools

import numpy as np
import jax
import jax.numpy as jnp
from jax import lax
from jax.experimental import pallas as pl
from jax.experimental.pallas import tpu as pltpu
from jax.experimental.pallas import tpu_sc as plsc

POS_THRESH = 0.1
NEG_THRESH = 1.4
NUM_POS = 5192
NUM_HN_SAMPLES = 2048

NW = 32
NS = 16
PCH = 88
NPCH = 2
PB = PCH * NPCH
M_PAD = NW * PB
BLK_M = 704
KROWS = 640
KBUF = KROWS + 32
CK_PAD = NS * KROWS + 512
KCHUNK = 512
SENT = 0x7FFFFFFF


@functools.lru_cache(maxsize=None)
def _selections(N0, N1, n_pairs):
    rng = np.random.RandomState(0)
    sel0 = rng.choice(N0, min(N0, NUM_HN_SAMPLES), replace=False)
    sel1 = rng.choice(N1, min(N1, NUM_HN_SAMPLES), replace=False)
    if n_pairs > NUM_POS:
        pos_sel = rng.choice(n_pairs, NUM_POS, replace=False)
    else:
        pos_sel = np.arange(n_pairs)
    return sel0.astype(np.int32), sel1.astype(np.int32), pos_sel.astype(np.int32)


def _sc_gather_fn(n_sub, n_pairs):
    sb = n_sub // NW
    mesh = plsc.VectorSubcoreMesh(core_axis_name="c", subcore_axis_name="s")
    out_type = [
        jax.ShapeDtypeStruct((M_PAD, 128), jnp.float32),
        jax.ShapeDtypeStruct((M_PAD, 128), jnp.float32),
        jax.ShapeDtypeStruct((n_sub, 128), jnp.float32),
        jax.ShapeDtypeStruct((n_sub, 128), jnp.float32),
        jax.ShapeDtypeStruct((NW, NPCH, PCH), jnp.int32),
        jax.ShapeDtypeStruct((NW, NPCH, PCH), jnp.int32),
        jax.ShapeDtypeStruct((CK_PAD,), jnp.int32),
        jax.ShapeDtypeStruct((CK_PAD,), jnp.int32),
        jax.ShapeDtypeStruct((2, 16), jnp.int32),
    ]
    scratch = [
        pltpu.VMEM((NPCH, PCH), jnp.int32),
        pltpu.VMEM((NPCH, PCH), jnp.int32),
        pltpu.VMEM((NPCH, PCH), jnp.int32),
        pltpu.VMEM((NPCH, PCH), jnp.int32),
        pltpu.VMEM((2 * NPCH, PCH, 128), jnp.float32),
        pltpu.VMEM((sb,), jnp.int32),
        pltpu.VMEM((sb,), jnp.int32),
        pltpu.VMEM((sb, 128), jnp.float32),
        pltpu.VMEM((sb, 128), jnp.float32),
        pltpu.VMEM((20000,), jnp.int32),
        pltpu.VMEM((2 * KROWS,), jnp.int32),
        pltpu.VMEM((KBUF,), jnp.int32),
        pltpu.VMEM((512,), jnp.int32),
        pltpu.VMEM((16,), jnp.int32),
        pltpu.VMEM((16, 16), jnp.int32),
        pltpu.VMEM_SHARED((16, 16), jnp.int32),
        pltpu.SemaphoreType.DMA,
        pltpu.SemaphoreType.DMA,
        pltpu.SemaphoreType.DMA,
        pltpu.SemaphoreType.DMA,
        pltpu.SemaphoreType.DMA,
        pltpu.SemaphoreType.DMA,
    ]

    @functools.partial(pl.kernel, mesh=mesh, out_type=out_type,
                       scratch_types=scratch,
                       compiler_params=pltpu.CompilerParams(
                           needs_layout_passes=False))
    def k(f0_h, f1_h, mflat_h, off0_h, off1_h, s0_h, s1_h, invT_h,
          posf0_o, posf1_o, subf0_o, subf1_o, pi0_o, pi1_o,
          ck0_o, ck1_o, cnt_o,
          off0v, off1v, pidx0v, pidx1v, prows, sidx0v, sidx1v,
          srows0, srows1, invbuf, mbuf, kbuf, sentbuf, cstage, cntv,
          shared_cnt, sem_i, sem_m, sem_s, sem_f, sem_o, sem_k):
        c = lax.axis_index("c")
        s = lax.axis_index("s")
        wid = s * 2 + c
        pbase = wid * PB
        sbase = wid * sb

        h_inv = pltpu.async_copy(invT_h.at[c], invbuf, sem_k)
        mstart = 2 * jnp.maximum(
            jnp.minimum(s * KROWS, n_pairs - KROWS), 0)
        h_mb = pltpu.async_copy(
            mflat_h.at[pl.ds(pl.multiple_of(mstart, 8), 2 * KROWS)],
            mbuf, sem_k)

        h_idx = [
            pltpu.async_copy(off0_h.at[wid], off0v, sem_i),
            pltpu.async_copy(off1_h.at[wid], off1v, sem_i),
            pltpu.async_copy(s0_h.at[wid], sidx0v, sem_i),
            pltpu.async_copy(s1_h.at[wid], sidx1v, sem_i),
        ]
        for h in h_idx:
            h.wait()

        h_m = []
        for cc in range(NPCH):
            h_m.append(pltpu.async_copy(mflat_h.at[off0v.at[cc]],
                                        pidx0v.at[cc], sem_m))
            h_m.append(pltpu.async_copy(mflat_h.at[off1v.at[cc]],
                                        pidx1v.at[cc], sem_m))
        h_s0 = pltpu.async_copy(f0_h.at[sidx0v], srows0, sem_s)
        h_s1 = pltpu.async_copy(f1_h.at[sidx1v], srows1, sem_s)
        for h in h_m:
            h.wait()

        h_f = []
        for cc in range(NPCH):
            h_f.append(pltpu.async_copy(f0_h.at[pidx0v.at[cc]],
                                        prows.at[cc], sem_f))
            h_f.append(pltpu.async_copy(f1_h.at[pidx1v.at[cc]],
                                        prows.at[NPCH + cc], sem_f))
        h_o = [
            pltpu.async_copy(pidx0v, pi0_o.at[wid], sem_o),
            pltpu.async_copy(pidx1v, pi1_o.at[wid], sem_o),
        ]

        h_inv.wait()
        h_mb.wait()
        rowbase = mstart // 2
        lane = lax.iota(jnp.int32, 16)
        sent16 = jnp.full((16,), SENT, jnp.int32)
        cur = jnp.int32(0)
        for i in range(KROWS // 16):
            pos16 = (i * 16 + lane) * 2
            kv = plsc.load_gather(mbuf, [pos16 + c])
            rv = plsc.load_gather(mbuf, [pos16 + (1 - c)])
            rk = plsc.load_gather(invbuf, [rv])
            rowv = rowbase + i * 16 + lane
            mask = (rk >= 0) & (rowv < n_pairs)
            key = kv * 2048 + rk
            _, cv = plsc.sort_key_val(jnp.where(mask, 0, 1), key)
            pc = jnp.max(plsc.all_reduce_population_count(mask))
            cstage[...] = cv
            off = cur & 15
            cur_a = pl.multiple_of(cur & ~jnp.int32(15), 16)
            lpos = (lane - off) & 15
            rot = plsc.load_gather(cstage, [lpos])
            w1 = (lane >= off) & (lpos < pc)
            w2 = (lane < off) & (lpos < pc)
            v1 = kbuf[pl.ds(cur_a, 16)]
            kbuf[pl.ds(cur_a, 16)] = jnp.where(w1, rot, v1)
            v2 = kbuf[pl.ds(cur_a + 16, 16)]
            kbuf[pl.ds(cur_a + 16, 16)] = jnp.where(w2, rot, v2)
            cur = cur + pc
        off = cur & 15
        cur_a = pl.multiple_of(cur & ~jnp.int32(15), 16)
        vt = kbuf[pl.ds(cur_a, 16)]
        kbuf[pl.ds(cur_a, 16)] = jnp.where(lane >= off, sent16, vt)
        kbuf[pl.ds(cur_a + 16, 16)] = sent16
        rcnt = (cur + 7) & ~jnp.int32(7)

        cstage[...] = jnp.full((16,), rcnt, jnp.int32)
        pltpu.sync_copy(cstage, shared_cnt.at[s])

        h_s0.wait()
        h_s1.wait()
        h_o.append(pltpu.async_copy(srows0, subf0_o.at[pl.ds(sbase, sb)], sem_o))
        h_o.append(pltpu.async_copy(srows1, subf1_o.at[pl.ds(sbase, sb)], sem_o))
        for h in h_f:
            h.wait()
        for cc in range(NPCH):
            h_o.append(pltpu.async_copy(
                prows.at[cc], posf0_o.at[pl.ds(pbase + cc * PCH, PCH)], sem_o))
            h_o.append(pltpu.async_copy(
                prows.at[NPCH + cc], posf1_o.at[pl.ds(pbase + cc * PCH, PCH)],
                sem_o))

        plsc.subcore_barrier()
        pltpu.sync_copy(shared_cnt, cntv)
        rcnts = plsc.load_gather(cntv, [lane, jnp.zeros((16,), jnp.int32)])
        offset = jnp.sum(jnp.where(lane < s, rcnts, 0))
        total = jnp.sum(rcnts)

        def emit_copies(ck_o):
            for bit in (512, 256, 128, 64, 32, 16, 8):
                srcoff = rcnt & ~jnp.int32(2 * bit - 1)

                @pl.when((rcnt & bit) != 0)
                def _copy(bit=bit, srcoff=srcoff):
                    pltpu.sync_copy(
                        kbuf.at[pl.ds(pl.multiple_of(srcoff, 8), bit)],
                        ck_o.at[pl.ds(pl.multiple_of(offset + srcoff, 8),
                                      bit)])

            @pl.when(s == 0)
            def _tail():
                for j in range(32):
                    sentbuf[pl.ds(j * 16, 16)] = jnp.full((16,), SENT,
                                                          jnp.int32)
                pltpu.sync_copy(sentbuf,
                                ck_o.at[pl.ds(pl.multiple_of(total, 8), 512)])
                cstage[...] = jnp.full((16,), total, jnp.int32)
                pltpu.sync_copy(cstage, cnt_o.at[c])

        @pl.when(c == 0)
        def _side0():
            emit_copies(ck0_o)

        @pl.when(c == 1)
        def _side1():
            emit_copies(ck1_o)

        for h in h_o:
            h.wait()

    return k


def _tc_loss_kernel(posf0_ref, posf1_ref, subf0_ref, subf1_ref,
                    pi0_ref, pi1_ref, ck0_ref, ck1_ref, cnt_ref,
                    out_ref, acc_ref, ba0_ref, ba1_ref, m0_ref, m1_ref,
                    *, n_valid, n_sub, grid_m):
    p = pl.program_id(0)

    a0 = posf0_ref[...]
    a1 = posf1_ref[...]

    dotb = functools.partial(
        lax.dot_general,
        dimension_numbers=(((1,), (1,)), ((), ())),
        preferred_element_type=jnp.float32,
    )

    def augment(x, col1, col2):
        return jnp.concatenate([x, col1, col2], axis=1).astype(jnp.bfloat16)

    @pl.when(p == 0)
    def _init():
        for i in range(5):
            acc_ref[i] = 0.0
        b0 = subf0_ref[...]
        b1 = subf1_ref[...]
        onesb = jnp.ones((n_sub, 1), jnp.float32)
        ba0_ref[...] = augment(-2.0 * b0,
                               jnp.sum(b0 * b0, axis=1, keepdims=True), onesb)
        ba1_ref[...] = augment(-2.0 * b1,
                               jnp.sum(b1 * b1, axis=1, keepdims=True), onesb)

    a0sq = jnp.sum(a0 * a0, axis=1, keepdims=True)
    a1sq = jnp.sum(a1 * a1, axis=1, keepdims=True)
    onesa = jnp.ones((BLK_M, 1), jnp.float32)
    aa0 = augment(a0, onesa, a0sq)
    aa1 = augment(a1, onesa, a1sq)

    rows = p * BLK_M + lax.broadcasted_iota(jnp.int32, (BLK_M, 1), 0)
    valid = rows < n_valid
    jrow = lax.broadcasted_iota(jnp.int32, (BLK_M, n_sub), 1)

    def side(aaug, ba_ref):
        d2 = jnp.maximum(dotb(aaug, ba_ref[...]), 0.0)
        bc = lax.bitcast_convert_type(d2, jnp.int32)
        key = jnp.bitwise_or(jnp.bitwise_and(bc, jnp.int32(~2047)), jrow)
        kmin = jnp.min(key, axis=1, keepdims=True)
        rank = jnp.bitwise_and(kmin, 2047)
        dmin = lax.bitcast_convert_type(
            jnp.bitwise_and(kmin, jnp.int32(~2047)), jnp.float32)
        dist = jnp.sqrt(dmin + 1e-07)
        nl = jnp.square(jnp.maximum(NEG_THRESH - dist, 0.0))
        return nl, rank

    nl0, rank0 = side(aa0, ba1_ref)
    nl1, rank1 = side(aa1, ba0_ref)

    q0 = pi0_ref[...] * 2048 + rank0
    q1 = pi1_ref[...] * 2048 + rank1
    cnt0 = cnt_ref[0, 0]
    cnt1 = cnt_ref[1, 0]

    m0_ref[...] = jnp.full((BLK_M, 1), SENT, jnp.int32)
    m1_ref[...] = jnp.full((BLK_M, 1), SENT, jnp.int32)
    for ci in range((CK_PAD - 512) // KCHUNK):
        @pl.when(ci * KCHUNK < cnt0)
        def _c0(ci=ci):
            ch = ck0_ref[:, pl.ds(ci * KCHUNK, KCHUNK)]
            x = jnp.min(jnp.bitwise_xor(q0, ch), axis=1, keepdims=True)
            m0_ref[...] = jnp.minimum(m0_ref[...], x)

        @pl.when(ci * KCHUNK < cnt1)
        def _c1(ci=ci):
            ch = ck1_ref[:, pl.ds(ci * KCHUNK, KCHUNK)]
            x = jnp.min(jnp.bitwise_xor(q1, ch), axis=1, keepdims=True)
            m1_ref[...] = jnp.minimum(m1_ref[...], x)

    mask0 = valid & (m0_ref[...] != 0)
    mask1 = valid & (m1_ref[...] != 0)

    dpos = a0 - a1
    pos_sq = jnp.sum(dpos * dpos, axis=1, keepdims=True)
    pos_term = jnp.where(valid, jnp.maximum(pos_sq - POS_THRESH, 0.0), 0.0)

    acc_ref[0] += jnp.sum(pos_term)
    acc_ref[1] += jnp.sum(jnp.where(mask0, nl0, 0.0))
    acc_ref[2] += jnp.sum(mask0.astype(jnp.float32))
    acc_ref[3] += jnp.sum(jnp.where(mask1, nl1, 0.0))
    acc_ref[4] += jnp.sum(mask1.astype(jnp.float32))

    @pl.when(p == grid_m - 1)
    def _fin():
        pos_loss = acc_ref[0] / n_valid
        neg0 = acc_ref[1] / jnp.maximum(acc_ref[2], 1.0)
        neg1 = acc_ref[3] / jnp.maximum(acc_ref[4], 1.0)
        out_ref[0, 0] = pos_loss + (neg0 + neg1) / 2.0


def _tc_loss(posF0, posF1, subF0, subF1, pi0, pi1, ck0, ck1, cnts, n_valid):
    n_sub = subF0.shape[0]
    grid_m = M_PAD // BLK_M
    kern = functools.partial(
        _tc_loss_kernel, n_valid=n_valid, n_sub=n_sub, grid_m=grid_m)
    full = lambda shape: pl.BlockSpec(shape, lambda p: (0, 0))
    out = pl.pallas_call(
        kern,
        grid=(grid_m,),
        in_specs=[
            pl.BlockSpec((BLK_M, 128), lambda p: (p, 0)),
            pl.BlockSpec((BLK_M, 128), lambda p: (p, 0)),
            full((n_sub, 128)),
            full((n_sub, 128)),
            pl.BlockSpec((BLK_M, 1), lambda p: (p, 0)),
            pl.BlockSpec((BLK_M, 1), lambda p: (p, 0)),
            full((1, CK_PAD)),
            full((1, CK_PAD)),
            pl.BlockSpec(memory_space=pltpu.SMEM),
        ],
        out_specs=pl.BlockSpec(memory_space=pltpu.SMEM),
        out_shape=jax.ShapeDtypeStruct((1, 1), jnp.float32),
        scratch_shapes=[
            pltpu.SMEM((8,), jnp.float32),
            pltpu.VMEM((n_sub, 130), jnp.bfloat16),
            pltpu.VMEM((n_sub, 130), jnp.bfloat16),
            pltpu.VMEM((BLK_M, 1), jnp.int32),
            pltpu.VMEM((BLK_M, 1), jnp.int32),
        ],
        compiler_params=pltpu.CompilerParams(
            dimension_semantics=("arbitrary",)),
    )(posF0, posF1, subF0, subF1, pi0, pi1, ck0, ck1, cnts)
    return out[0, 0]


def kernel(F0, F1, matches):
    N0, N1 = int(F0.shape[0]), int(F1.shape[0])
    n_pairs = int(matches.shape[0])
    sel0, sel1, pos_sel = _selections(N0, N1, n_pairs)
    n_valid = len(pos_sel)
    n_sub = len(sel0)

    pos_pad = np.zeros(M_PAD, np.int32)
    pos_pad[:n_valid] = pos_sel
    off0 = (2 * pos_pad).reshape(NW, NPCH, PCH)
    off1 = (2 * pos_pad + 1).reshape(NW, NPCH, PCH)
    s0w = sel0.reshape(NW, n_sub // NW)
    s1w = sel1.reshape(NW, n_sub // NW)
    invT = np.full((2, N0), -1, np.int32)
    invT[0, sel1] = np.arange(n_sub, dtype=np.int32)
    invT[1, sel0] = np.arange(n_sub, dtype=np.int32)

    matches = matches.astype(jnp.int32)
    mflat = matches.reshape(-1)

    (posF0, posF1, subF0, subF1, pi0, pi1,
     ck0, ck1, cnts) = _sc_gather_fn(n_sub, n_pairs)(
        F0, F1, mflat,
        jnp.asarray(off0), jnp.asarray(off1),
        jnp.asarray(s0w), jnp.asarray(s1w), jnp.asarray(invT))

    pi0 = pi0.reshape(M_PAD, 1)
    pi1 = pi1.reshape(M_PAD, 1)
    ck0 = ck0.reshape(1, CK_PAD)
    ck1 = ck1.reshape(1, CK_PAD)

    return _tc_loss(posF0, posF1, subF0, subF1, pi0, pi1, ck0, ck1,
                    cnts, n_valid)


# speedup vs baseline: 1.7594x; 1.1238x over previous
import functools

import numpy as np
import jax
import jax.numpy as jnp
from jax import lax
from jax.experimental import pallas as pl
from jax.experimental.pallas import tpu as pltpu
from jax.experimental.pallas import tpu_sc as plsc

POS_THRESH = 0.1
NEG_THRESH = 1.4
NUM_POS = 5192
NUM_HN_SAMPLES = 2048

NW = 32
NS = 16
PCH = 88
NPCH = 2
PB = PCH * NPCH
M_PAD = NW * PB
BLK_M = 704
KROWS = 640
KBUF = KROWS + 32
CK_PAD = NS * KROWS + 1024
KCHUNK = 1024
SENT = 0x7FFFFFFF


@functools.lru_cache(maxsize=None)
def _selections(N0, N1, n_pairs):
    rng = np.random.RandomState(0)
    sel0 = rng.choice(N0, min(N0, NUM_HN_SAMPLES), replace=False)
    sel1 = rng.choice(N1, min(N1, NUM_HN_SAMPLES), replace=False)
    if n_pairs > NUM_POS:
        pos_sel = rng.choice(n_pairs, NUM_POS, replace=False)
    else:
        pos_sel = np.arange(n_pairs)
    return sel0.astype(np.int32), sel1.astype(np.int32), pos_sel.astype(np.int32)


def _sc_gather_fn(n_sub, n_pairs):
    sb = n_sub // NW
    mesh = plsc.VectorSubcoreMesh(core_axis_name="c", subcore_axis_name="s")
    out_type = [
        jax.ShapeDtypeStruct((M_PAD, 128), jnp.float32),
        jax.ShapeDtypeStruct((M_PAD, 128), jnp.float32),
        jax.ShapeDtypeStruct((n_sub, 128), jnp.float32),
        jax.ShapeDtypeStruct((n_sub, 128), jnp.float32),
        jax.ShapeDtypeStruct((NW, NPCH, PCH), jnp.int32),
        jax.ShapeDtypeStruct((NW, NPCH, PCH), jnp.int32),
        jax.ShapeDtypeStruct((CK_PAD,), jnp.int32),
        jax.ShapeDtypeStruct((CK_PAD,), jnp.int32),
        jax.ShapeDtypeStruct((2, 16), jnp.int32),
    ]
    scratch = [
        pltpu.VMEM((NPCH, PCH), jnp.int32),
        pltpu.VMEM((NPCH, PCH), jnp.int32),
        pltpu.VMEM((NPCH, PCH), jnp.int32),
        pltpu.VMEM((NPCH, PCH), jnp.int32),
        pltpu.VMEM((2 * NPCH, PCH, 128), jnp.float32),
        pltpu.VMEM((sb,), jnp.int32),
        pltpu.VMEM((sb,), jnp.int32),
        pltpu.VMEM((sb, 128), jnp.float32),
        pltpu.VMEM((sb, 128), jnp.float32),
        pltpu.VMEM((20000,), jnp.int32),
        pltpu.VMEM((2 * KROWS,), jnp.int32),
        pltpu.VMEM((KBUF,), jnp.int32),
        pltpu.VMEM((512,), jnp.int32),
        pltpu.VMEM((16,), jnp.int32),
        pltpu.VMEM((16, 16), jnp.int32),
        pltpu.VMEM_SHARED((16, 16), jnp.int32),
        pltpu.SemaphoreType.DMA,
        pltpu.SemaphoreType.DMA,
        pltpu.SemaphoreType.DMA,
        pltpu.SemaphoreType.DMA,
        pltpu.SemaphoreType.DMA,
        pltpu.SemaphoreType.DMA,
    ]

    @functools.partial(pl.kernel, mesh=mesh, out_type=out_type,
                       scratch_types=scratch,
                       compiler_params=pltpu.CompilerParams(
                           needs_layout_passes=False))
    def k(f0_h, f1_h, mflat_h, off0_h, off1_h, s0_h, s1_h, invT_h,
          posf0_o, posf1_o, subf0_o, subf1_o, pi0_o, pi1_o,
          ck0_o, ck1_o, cnt_o,
          off0v, off1v, pidx0v, pidx1v, prows, sidx0v, sidx1v,
          srows0, srows1, invbuf, mbuf, kbuf, sentbuf, cstage, cntv,
          shared_cnt, sem_i, sem_m, sem_s, sem_f, sem_o, sem_k):
        c = lax.axis_index("c")
        s = lax.axis_index("s")
        wid = s * 2 + c
        pbase = wid * PB
        sbase = wid * sb

        h_inv = pltpu.async_copy(invT_h.at[c], invbuf, sem_k)
        mstart = 2 * jnp.maximum(
            jnp.minimum(s * KROWS, n_pairs - KROWS), 0)
        h_mb = pltpu.async_copy(
            mflat_h.at[pl.ds(pl.multiple_of(mstart, 8), 2 * KROWS)],
            mbuf, sem_k)

        h_idx = [
            pltpu.async_copy(off0_h.at[wid], off0v, sem_i),
            pltpu.async_copy(off1_h.at[wid], off1v, sem_i),
            pltpu.async_copy(s0_h.at[wid], sidx0v, sem_i),
            pltpu.async_copy(s1_h.at[wid], sidx1v, sem_i),
        ]
        for h in h_idx:
            h.wait()

        h_m = []
        for cc in range(NPCH):
            h_m.append(pltpu.async_copy(mflat_h.at[off0v.at[cc]],
                                        pidx0v.at[cc], sem_m))
            h_m.append(pltpu.async_copy(mflat_h.at[off1v.at[cc]],
                                        pidx1v.at[cc], sem_m))
        h_s0 = pltpu.async_copy(f0_h.at[sidx0v], srows0, sem_s)
        h_s1 = pltpu.async_copy(f1_h.at[sidx1v], srows1, sem_s)
        for h in h_m:
            h.wait()

        h_f = []
        for cc in range(NPCH):
            h_f.append(pltpu.async_copy(f0_h.at[pidx0v.at[cc]],
                                        prows.at[cc], sem_f))
            h_f.append(pltpu.async_copy(f1_h.at[pidx1v.at[cc]],
                                        prows.at[NPCH + cc], sem_f))
        h_o = [
            pltpu.async_copy(pidx0v, pi0_o.at[wid], sem_o),
            pltpu.async_copy(pidx1v, pi1_o.at[wid], sem_o),
        ]

        h_inv.wait()
        h_mb.wait()
        rowbase = mstart // 2
        lane = lax.iota(jnp.int32, 16)
        sent16 = jnp.full((16,), SENT, jnp.int32)
        cur = jnp.int32(0)
        for i in range(KROWS // 16):
            pos16 = (i * 16 + lane) * 2
            kv = plsc.load_gather(mbuf, [pos16 + c])
            rv = plsc.load_gather(mbuf, [pos16 + (1 - c)])
            rk = plsc.load_gather(invbuf, [rv])
            rowv = rowbase + i * 16 + lane
            mask = (rk >= 0) & (rowv < n_pairs)
            key = kv * 2048 + rk
            _, cv = plsc.sort_key_val(jnp.where(mask, 0, 1), key)
            pc = jnp.max(plsc.all_reduce_population_count(mask))
            cstage[...] = cv
            off = cur & 15
            cur_a = pl.multiple_of(cur & ~jnp.int32(15), 16)
            lpos = (lane - off) & 15
            rot = plsc.load_gather(cstage, [lpos])
            w1 = (lane >= off) & (lpos < pc)
            w2 = (lane < off) & (lpos < pc)
            v1 = kbuf[pl.ds(cur_a, 16)]
            kbuf[pl.ds(cur_a, 16)] = jnp.where(w1, rot, v1)
            v2 = kbuf[pl.ds(cur_a + 16, 16)]
            kbuf[pl.ds(cur_a + 16, 16)] = jnp.where(w2, rot, v2)
            cur = cur + pc
        off = cur & 15
        cur_a = pl.multiple_of(cur & ~jnp.int32(15), 16)
        vt = kbuf[pl.ds(cur_a, 16)]
        kbuf[pl.ds(cur_a, 16)] = jnp.where(lane >= off, sent16, vt)
        kbuf[pl.ds(cur_a + 16, 16)] = sent16
        rcnt = (cur + 7) & ~jnp.int32(7)

        cstage[...] = jnp.full((16,), rcnt, jnp.int32)
        pltpu.sync_copy(cstage, shared_cnt.at[s])

        h_s0.wait()
        h_s1.wait()
        h_o.append(pltpu.async_copy(srows0, subf0_o.at[pl.ds(sbase, sb)], sem_o))
        h_o.append(pltpu.async_copy(srows1, subf1_o.at[pl.ds(sbase, sb)], sem_o))
        for h in h_f:
            h.wait()
        for cc in range(NPCH):
            h_o.append(pltpu.async_copy(
                prows.at[cc], posf0_o.at[pl.ds(pbase + cc * PCH, PCH)], sem_o))
            h_o.append(pltpu.async_copy(
                prows.at[NPCH + cc], posf1_o.at[pl.ds(pbase + cc * PCH, PCH)],
                sem_o))

        plsc.subcore_barrier()
        pltpu.sync_copy(shared_cnt, cntv)
        rcnts = plsc.load_gather(cntv, [lane, jnp.zeros((16,), jnp.int32)])
        offset = jnp.sum(jnp.where(lane < s, rcnts, 0))
        total = jnp.sum(rcnts)

        def emit_copies(ck_o):
            for bit in (512, 256, 128, 64, 32, 16, 8):
                srcoff = rcnt & ~jnp.int32(2 * bit - 1)

                @pl.when((rcnt & bit) != 0)
                def _copy(bit=bit, srcoff=srcoff):
                    pltpu.sync_copy(
                        kbuf.at[pl.ds(pl.multiple_of(srcoff, 8), bit)],
                        ck_o.at[pl.ds(pl.multiple_of(offset + srcoff, 8),
                                      bit)])

            @pl.when(s == 0)
            def _tail():
                for j in range(32):
                    sentbuf[pl.ds(j * 16, 16)] = jnp.full((16,), SENT,
                                                          jnp.int32)
                pltpu.sync_copy(sentbuf,
                                ck_o.at[pl.ds(pl.multiple_of(total, 8), 512)])
                pltpu.sync_copy(
                    sentbuf,
                    ck_o.at[pl.ds(pl.multiple_of(total + 512, 8), 512)])
                cstage[...] = jnp.full((16,), total, jnp.int32)
                pltpu.sync_copy(cstage, cnt_o.at[c])

        @pl.when(c == 0)
        def _side0():
            emit_copies(ck0_o)

        @pl.when(c == 1)
        def _side1():
            emit_copies(ck1_o)

        for h in h_o:
            h.wait()

    return k


def _tc_loss_kernel(posf0_ref, posf1_ref, subf0_ref, subf1_ref,
                    pi0_ref, pi1_ref, ck0_ref, ck1_ref, cnt_ref,
                    out_ref, acc_ref, ba0_ref, ba1_ref, m0_ref, m1_ref,
                    *, n_valid, n_sub, grid_m):
    p = pl.program_id(0)

    a0 = posf0_ref[...]
    a1 = posf1_ref[...]

    dotb = functools.partial(
        lax.dot_general,
        dimension_numbers=(((1,), (1,)), ((), ())),
        preferred_element_type=jnp.float32,
    )

    def augment(x, col1, col2):
        return jnp.concatenate([x, col1, col2], axis=1).astype(jnp.bfloat16)

    @pl.when(p == 0)
    def _init():
        for i in range(5):
            acc_ref[i] = 0.0
        b0 = subf0_ref[...]
        b1 = subf1_ref[...]
        onesb = jnp.ones((n_sub, 1), jnp.float32)
        ba0_ref[...] = augment(-2.0 * b0,
                               jnp.sum(b0 * b0, axis=1, keepdims=True), onesb)
        ba1_ref[...] = augment(-2.0 * b1,
                               jnp.sum(b1 * b1, axis=1, keepdims=True), onesb)

    a0sq = jnp.sum(a0 * a0, axis=1, keepdims=True)
    a1sq = jnp.sum(a1 * a1, axis=1, keepdims=True)
    onesa = jnp.ones((BLK_M, 1), jnp.float32)
    aa0 = augment(a0, onesa, a0sq)
    aa1 = augment(a1, onesa, a1sq)

    rows = p * BLK_M + lax.broadcasted_iota(jnp.int32, (BLK_M, 1), 0)
    valid = rows < n_valid
    jrow = lax.broadcasted_iota(jnp.int32, (BLK_M, n_sub), 1)

    def side(aaug, ba_ref):
        d2 = jnp.maximum(dotb(aaug, ba_ref[...]), 0.0)
        bc = lax.bitcast_convert_type(d2, jnp.int32)
        key = jnp.bitwise_or(jnp.bitwise_and(bc, jnp.int32(~2047)), jrow)
        kmin = jnp.min(key, axis=1, keepdims=True)
        rank = jnp.bitwise_and(kmin, 2047)
        dmin = lax.bitcast_convert_type(
            jnp.bitwise_and(kmin, jnp.int32(~2047)), jnp.float32)
        dist = jnp.sqrt(dmin + 1e-07)
        nl = jnp.square(jnp.maximum(NEG_THRESH - dist, 0.0))
        return nl, rank

    nl0, rank0 = side(aa0, ba1_ref)
    nl1, rank1 = side(aa1, ba0_ref)

    q0 = pi0_ref[...] * 2048 + rank0
    q1 = pi1_ref[...] * 2048 + rank1
    cnt0 = cnt_ref[0, 0]
    cnt1 = cnt_ref[1, 0]

    m0_ref[...] = jnp.full((BLK_M, 1), SENT, jnp.int32)
    m1_ref[...] = jnp.full((BLK_M, 1), SENT, jnp.int32)
    for ci in range((CK_PAD - 1024) // KCHUNK):
        @pl.when(ci * KCHUNK < cnt0)
        def _c0(ci=ci):
            ch = ck0_ref[:, pl.ds(ci * KCHUNK, KCHUNK)]
            x = jnp.min(jnp.bitwise_xor(q0, ch), axis=1, keepdims=True)
            m0_ref[...] = jnp.minimum(m0_ref[...], x)

        @pl.when(ci * KCHUNK < cnt1)
        def _c1(ci=ci):
            ch = ck1_ref[:, pl.ds(ci * KCHUNK, KCHUNK)]
            x = jnp.min(jnp.bitwise_xor(q1, ch), axis=1, keepdims=True)
            m1_ref[...] = jnp.minimum(m1_ref[...], x)

    mask0 = valid & (m0_ref[...] != 0)
    mask1 = valid & (m1_ref[...] != 0)

    dpos = a0 - a1
    pos_sq = jnp.sum(dpos * dpos, axis=1, keepdims=True)
    pos_term = jnp.where(valid, jnp.maximum(pos_sq - POS_THRESH, 0.0), 0.0)

    acc_ref[0] += jnp.sum(pos_term)
    acc_ref[1] += jnp.sum(jnp.where(mask0, nl0, 0.0))
    acc_ref[2] += jnp.sum(mask0.astype(jnp.float32))
    acc_ref[3] += jnp.sum(jnp.where(mask1, nl1, 0.0))
    acc_ref[4] += jnp.sum(mask1.astype(jnp.float32))

    @pl.when(p == grid_m - 1)
    def _fin():
        pos_loss = acc_ref[0] / n_valid
        neg0 = acc_ref[1] / jnp.maximum(acc_ref[2], 1.0)
        neg1 = acc_ref[3] / jnp.maximum(acc_ref[4], 1.0)
        out_ref[0, 0] = pos_loss + (neg0 + neg1) / 2.0


def _tc_loss(posF0, posF1, subF0, subF1, pi0, pi1, ck0, ck1, cnts, n_valid):
    n_sub = subF0.shape[0]
    grid_m = M_PAD // BLK_M
    kern = functools.partial(
        _tc_loss_kernel, n_valid=n_valid, n_sub=n_sub, grid_m=grid_m)
    full = lambda shape: pl.BlockSpec(shape, lambda p: (0, 0))
    out = pl.pallas_call(
        kern,
        grid=(grid_m,),
        in_specs=[
            pl.BlockSpec((BLK_M, 128), lambda p: (p, 0)),
            pl.BlockSpec((BLK_M, 128), lambda p: (p, 0)),
            full((n_sub, 128)),
            full((n_sub, 128)),
            pl.BlockSpec((BLK_M, 1), lambda p: (p, 0)),
            pl.BlockSpec((BLK_M, 1), lambda p: (p, 0)),
            full((1, CK_PAD)),
            full((1, CK_PAD)),
            pl.BlockSpec(memory_space=pltpu.SMEM),
        ],
        out_specs=pl.BlockSpec(memory_space=pltpu.SMEM),
        out_shape=jax.ShapeDtypeStruct((1, 1), jnp.float32),
        scratch_shapes=[
            pltpu.SMEM((8,), jnp.float32),
            pltpu.VMEM((n_sub, 130), jnp.bfloat16),
            pltpu.VMEM((n_sub, 130), jnp.bfloat16),
            pltpu.VMEM((BLK_M, 1), jnp.int32),
            pltpu.VMEM((BLK_M, 1), jnp.int32),
        ],
        compiler_params=pltpu.CompilerParams(
            dimension_semantics=("arbitrary",)),
    )(posF0, posF1, subF0, subF1, pi0, pi1, ck0, ck1, cnts)
    return out[0, 0]


def kernel(F0, F1, matches):
    N0, N1 = int(F0.shape[0]), int(F1.shape[0])
    n_pairs = int(matches.shape[0])
    sel0, sel1, pos_sel = _selections(N0, N1, n_pairs)
    n_valid = len(pos_sel)
    n_sub = len(sel0)

    pos_pad = np.zeros(M_PAD, np.int32)
    pos_pad[:n_valid] = pos_sel
    off0 = (2 * pos_pad).reshape(NW, NPCH, PCH)
    off1 = (2 * pos_pad + 1).reshape(NW, NPCH, PCH)
    s0w = sel0.reshape(NW, n_sub // NW)
    s1w = sel1.reshape(NW, n_sub // NW)
    invT = np.full((2, N0), -1, np.int32)
    invT[0, sel1] = np.arange(n_sub, dtype=np.int32)
    invT[1, sel0] = np.arange(n_sub, dtype=np.int32)

    matches = matches.astype(jnp.int32)
    mflat = matches.reshape(-1)

    (posF0, posF1, subF0, subF1, pi0, pi1,
     ck0, ck1, cnts) = _sc_gather_fn(n_sub, n_pairs)(
        F0, F1, mflat,
        jnp.asarray(off0), jnp.asarray(off1),
        jnp.asarray(s0w), jnp.asarray(s1w), jnp.asarray(invT))

    pi0 = pi0.reshape(M_PAD, 1)
    pi1 = pi1.reshape(M_PAD, 1)
    ck0 = ck0.reshape(1, CK_PAD)
    ck1 = ck1.reshape(1, CK_PAD)

    return _tc_loss(posF0, posF1, subF0, subF1, pi0, pi1, ck0, ck1,
                    cnts, n_valid)


# speedup vs baseline: 1.7752x; 1.0090x over previous
import functools

import numpy as np
import jax
import jax.numpy as jnp
from jax import lax
from jax.experimental import pallas as pl
from jax.experimental.pallas import tpu as pltpu
from jax.experimental.pallas import tpu_sc as plsc

POS_THRESH = 0.1
NEG_THRESH = 1.4
NUM_POS = 5192
NUM_HN_SAMPLES = 2048

NW = 32
NS = 16
PCH = 88
NPCH = 2
PB = PCH * NPCH
M_PAD = NW * PB
BLK_M = 704
KROWS = 640
KBUF = KROWS + 32
CK_PAD = NS * KROWS + 2048
KCHUNK = 2048
SENT = 0x7FFFFFFF


@functools.lru_cache(maxsize=None)
def _selections(N0, N1, n_pairs):
    rng = np.random.RandomState(0)
    sel0 = rng.choice(N0, min(N0, NUM_HN_SAMPLES), replace=False)
    sel1 = rng.choice(N1, min(N1, NUM_HN_SAMPLES), replace=False)
    if n_pairs > NUM_POS:
        pos_sel = rng.choice(n_pairs, NUM_POS, replace=False)
    else:
        pos_sel = np.arange(n_pairs)
    return sel0.astype(np.int32), sel1.astype(np.int32), pos_sel.astype(np.int32)


def _sc_gather_fn(n_sub, n_pairs):
    sb = n_sub // NW
    mesh = plsc.VectorSubcoreMesh(core_axis_name="c", subcore_axis_name="s")
    out_type = [
        jax.ShapeDtypeStruct((M_PAD, 128), jnp.float32),
        jax.ShapeDtypeStruct((M_PAD, 128), jnp.float32),
        jax.ShapeDtypeStruct((n_sub, 128), jnp.float32),
        jax.ShapeDtypeStruct((n_sub, 128), jnp.float32),
        jax.ShapeDtypeStruct((NW, NPCH, PCH), jnp.int32),
        jax.ShapeDtypeStruct((NW, NPCH, PCH), jnp.int32),
        jax.ShapeDtypeStruct((CK_PAD,), jnp.int32),
        jax.ShapeDtypeStruct((CK_PAD,), jnp.int32),
        jax.ShapeDtypeStruct((2, 16), jnp.int32),
    ]
    scratch = [
        pltpu.VMEM((NPCH, PCH), jnp.int32),
        pltpu.VMEM((NPCH, PCH), jnp.int32),
        pltpu.VMEM((NPCH, PCH), jnp.int32),
        pltpu.VMEM((NPCH, PCH), jnp.int32),
        pltpu.VMEM((2 * NPCH, PCH, 128), jnp.float32),
        pltpu.VMEM((sb,), jnp.int32),
        pltpu.VMEM((sb,), jnp.int32),
        pltpu.VMEM((sb, 128), jnp.float32),
        pltpu.VMEM((sb, 128), jnp.float32),
        pltpu.VMEM((20000,), jnp.int32),
        pltpu.VMEM((2 * KROWS,), jnp.int32),
        pltpu.VMEM((KBUF,), jnp.int32),
        pltpu.VMEM((512,), jnp.int32),
        pltpu.VMEM((16,), jnp.int32),
        pltpu.VMEM((16, 16), jnp.int32),
        pltpu.VMEM_SHARED((16, 16), jnp.int32),
        pltpu.SemaphoreType.DMA,
        pltpu.SemaphoreType.DMA,
        pltpu.SemaphoreType.DMA,
        pltpu.SemaphoreType.DMA,
        pltpu.SemaphoreType.DMA,
        pltpu.SemaphoreType.DMA,
    ]

    @functools.partial(pl.kernel, mesh=mesh, out_type=out_type,
                       scratch_types=scratch,
                       compiler_params=pltpu.CompilerParams(
                           needs_layout_passes=False))
    def k(f0_h, f1_h, mflat_h, off0_h, off1_h, s0_h, s1_h, invT_h,
          posf0_o, posf1_o, subf0_o, subf1_o, pi0_o, pi1_o,
          ck0_o, ck1_o, cnt_o,
          off0v, off1v, pidx0v, pidx1v, prows, sidx0v, sidx1v,
          srows0, srows1, invbuf, mbuf, kbuf, sentbuf, cstage, cntv,
          shared_cnt, sem_i, sem_m, sem_s, sem_f, sem_o, sem_k):
        c = lax.axis_index("c")
        s = lax.axis_index("s")
        wid = s * 2 + c
        pbase = wid * PB
        sbase = wid * sb

        h_inv = pltpu.async_copy(invT_h.at[c], invbuf, sem_k)
        mstart = 2 * jnp.maximum(
            jnp.minimum(s * KROWS, n_pairs - KROWS), 0)
        h_mb = pltpu.async_copy(
            mflat_h.at[pl.ds(pl.multiple_of(mstart, 8), 2 * KROWS)],
            mbuf, sem_k)

        h_idx = [
            pltpu.async_copy(off0_h.at[wid], off0v, sem_i),
            pltpu.async_copy(off1_h.at[wid], off1v, sem_i),
            pltpu.async_copy(s0_h.at[wid], sidx0v, sem_i),
            pltpu.async_copy(s1_h.at[wid], sidx1v, sem_i),
        ]
        for h in h_idx:
            h.wait()

        h_m = []
        for cc in range(NPCH):
            h_m.append(pltpu.async_copy(mflat_h.at[off0v.at[cc]],
                                        pidx0v.at[cc], sem_m))
            h_m.append(pltpu.async_copy(mflat_h.at[off1v.at[cc]],
                                        pidx1v.at[cc], sem_m))
        h_s0 = pltpu.async_copy(f0_h.at[sidx0v], srows0, sem_s)
        h_s1 = pltpu.async_copy(f1_h.at[sidx1v], srows1, sem_s)
        for h in h_m:
            h.wait()

        h_f = []
        for cc in range(NPCH):
            h_f.append(pltpu.async_copy(f0_h.at[pidx0v.at[cc]],
                                        prows.at[cc], sem_f))
            h_f.append(pltpu.async_copy(f1_h.at[pidx1v.at[cc]],
                                        prows.at[NPCH + cc], sem_f))
        h_o = [
            pltpu.async_copy(pidx0v, pi0_o.at[wid], sem_o),
            pltpu.async_copy(pidx1v, pi1_o.at[wid], sem_o),
        ]

        h_inv.wait()
        h_mb.wait()
        rowbase = mstart // 2
        lane = lax.iota(jnp.int32, 16)
        sent16 = jnp.full((16,), SENT, jnp.int32)
        cur = jnp.int32(0)
        for i in range(KROWS // 16):
            pos16 = (i * 16 + lane) * 2
            kv = plsc.load_gather(mbuf, [pos16 + c])
            rv = plsc.load_gather(mbuf, [pos16 + (1 - c)])
            rk = plsc.load_gather(invbuf, [rv])
            rowv = rowbase + i * 16 + lane
            mask = (rk >= 0) & (rowv < n_pairs)
            key = kv * 2048 + rk
            _, cv = plsc.sort_key_val(jnp.where(mask, 0, 1), key)
            pc = jnp.max(plsc.all_reduce_population_count(mask))
            cstage[...] = cv
            off = cur & 15
            cur_a = pl.multiple_of(cur & ~jnp.int32(15), 16)
            lpos = (lane - off) & 15
            rot = plsc.load_gather(cstage, [lpos])
            w1 = (lane >= off) & (lpos < pc)
            w2 = (lane < off) & (lpos < pc)
            v1 = kbuf[pl.ds(cur_a, 16)]
            kbuf[pl.ds(cur_a, 16)] = jnp.where(w1, rot, v1)
            v2 = kbuf[pl.ds(cur_a + 16, 16)]
            kbuf[pl.ds(cur_a + 16, 16)] = jnp.where(w2, rot, v2)
            cur = cur + pc
        off = cur & 15
        cur_a = pl.multiple_of(cur & ~jnp.int32(15), 16)
        vt = kbuf[pl.ds(cur_a, 16)]
        kbuf[pl.ds(cur_a, 16)] = jnp.where(lane >= off, sent16, vt)
        kbuf[pl.ds(cur_a + 16, 16)] = sent16
        rcnt = (cur + 7) & ~jnp.int32(7)

        cstage[...] = jnp.full((16,), rcnt, jnp.int32)
        pltpu.sync_copy(cstage, shared_cnt.at[s])

        h_s0.wait()
        h_s1.wait()
        h_o.append(pltpu.async_copy(srows0, subf0_o.at[pl.ds(sbase, sb)], sem_o))
        h_o.append(pltpu.async_copy(srows1, subf1_o.at[pl.ds(sbase, sb)], sem_o))
        for h in h_f:
            h.wait()
        for cc in range(NPCH):
            h_o.append(pltpu.async_copy(
                prows.at[cc], posf0_o.at[pl.ds(pbase + cc * PCH, PCH)], sem_o))
            h_o.append(pltpu.async_copy(
                prows.at[NPCH + cc], posf1_o.at[pl.ds(pbase + cc * PCH, PCH)],
                sem_o))

        plsc.subcore_barrier()
        pltpu.sync_copy(shared_cnt, cntv)
        rcnts = plsc.load_gather(cntv, [lane, jnp.zeros((16,), jnp.int32)])
        offset = jnp.sum(jnp.where(lane < s, rcnts, 0))
        total = jnp.sum(rcnts)

        def emit_copies(ck_o):
            for bit in (512, 256, 128, 64, 32, 16, 8):
                srcoff = rcnt & ~jnp.int32(2 * bit - 1)

                @pl.when((rcnt & bit) != 0)
                def _copy(bit=bit, srcoff=srcoff):
                    pltpu.sync_copy(
                        kbuf.at[pl.ds(pl.multiple_of(srcoff, 8), bit)],
                        ck_o.at[pl.ds(pl.multiple_of(offset + srcoff, 8),
                                      bit)])

            @pl.when(s == 0)
            def _tail():
                for j in range(32):
                    sentbuf[pl.ds(j * 16, 16)] = jnp.full((16,), SENT,
                                                          jnp.int32)
                for sj in range(4):
                    pltpu.sync_copy(
                        sentbuf,
                        ck_o.at[pl.ds(pl.multiple_of(total + sj * 512, 8),
                                      512)])
                cstage[...] = jnp.full((16,), total, jnp.int32)
                pltpu.sync_copy(cstage, cnt_o.at[c])

        @pl.when(c == 0)
        def _side0():
            emit_copies(ck0_o)

        @pl.when(c == 1)
        def _side1():
            emit_copies(ck1_o)

        for h in h_o:
            h.wait()

    return k


def _tc_loss_kernel(posf0_ref, posf1_ref, subf0_ref, subf1_ref,
                    pi0_ref, pi1_ref, ck0_ref, ck1_ref, cnt_ref,
                    out_ref, acc_ref, ba0_ref, ba1_ref, m0_ref, m1_ref,
                    *, n_valid, n_sub, grid_m):
    p = pl.program_id(0)

    a0 = posf0_ref[...]
    a1 = posf1_ref[...]

    dotb = functools.partial(
        lax.dot_general,
        dimension_numbers=(((1,), (1,)), ((), ())),
        preferred_element_type=jnp.float32,
    )

    def augment(x, col1, col2):
        return jnp.concatenate([x, col1, col2], axis=1).astype(jnp.bfloat16)

    @pl.when(p == 0)
    def _init():
        for i in range(5):
            acc_ref[i] = 0.0
        b0 = subf0_ref[...]
        b1 = subf1_ref[...]
        onesb = jnp.ones((n_sub, 1), jnp.float32)
        ba0_ref[...] = augment(-2.0 * b0,
                               jnp.sum(b0 * b0, axis=1, keepdims=True), onesb)
        ba1_ref[...] = augment(-2.0 * b1,
                               jnp.sum(b1 * b1, axis=1, keepdims=True), onesb)

    a0sq = jnp.sum(a0 * a0, axis=1, keepdims=True)
    a1sq = jnp.sum(a1 * a1, axis=1, keepdims=True)
    onesa = jnp.ones((BLK_M, 1), jnp.float32)
    aa0 = augment(a0, onesa, a0sq)
    aa1 = augment(a1, onesa, a1sq)

    rows = p * BLK_M + lax.broadcasted_iota(jnp.int32, (BLK_M, 1), 0)
    valid = rows < n_valid
    jrow = lax.broadcasted_iota(jnp.int32, (BLK_M, n_sub), 1)

    def side(aaug, ba_ref):
        d2 = jnp.maximum(dotb(aaug, ba_ref[...]), 0.0)
        bc = lax.bitcast_convert_type(d2, jnp.int32)
        key = jnp.bitwise_or(jnp.bitwise_and(bc, jnp.int32(~2047)), jrow)
        kmin = jnp.min(key, axis=1, keepdims=True)
        rank = jnp.bitwise_and(kmin, 2047)
        dmin = lax.bitcast_convert_type(
            jnp.bitwise_and(kmin, jnp.int32(~2047)), jnp.float32)
        dist = jnp.sqrt(dmin + 1e-07)
        nl = jnp.square(jnp.maximum(NEG_THRESH - dist, 0.0))
        return nl, rank

    nl0, rank0 = side(aa0, ba1_ref)
    nl1, rank1 = side(aa1, ba0_ref)

    q0 = pi0_ref[...] * 2048 + rank0
    q1 = pi1_ref[...] * 2048 + rank1
    cnt0 = cnt_ref[0, 0]
    cnt1 = cnt_ref[1, 0]

    m0_ref[...] = jnp.full((BLK_M, 1), SENT, jnp.int32)
    m1_ref[...] = jnp.full((BLK_M, 1), SENT, jnp.int32)
    for ci in range((CK_PAD - 2048) // KCHUNK):
        @pl.when(ci * KCHUNK < cnt0)
        def _c0(ci=ci):
            ch = ck0_ref[:, pl.ds(ci * KCHUNK, KCHUNK)]
            x = jnp.min(jnp.bitwise_xor(q0, ch), axis=1, keepdims=True)
            m0_ref[...] = jnp.minimum(m0_ref[...], x)

        @pl.when(ci * KCHUNK < cnt1)
        def _c1(ci=ci):
            ch = ck1_ref[:, pl.ds(ci * KCHUNK, KCHUNK)]
            x = jnp.min(jnp.bitwise_xor(q1, ch), axis=1, keepdims=True)
            m1_ref[...] = jnp.minimum(m1_ref[...], x)

    mask0 = valid & (m0_ref[...] != 0)
    mask1 = valid & (m1_ref[...] != 0)

    dpos = a0 - a1
    pos_sq = jnp.sum(dpos * dpos, axis=1, keepdims=True)
    pos_term = jnp.where(valid, jnp.maximum(pos_sq - POS_THRESH, 0.0), 0.0)

    acc_ref[0] += jnp.sum(pos_term)
    acc_ref[1] += jnp.sum(jnp.where(mask0, nl0, 0.0))
    acc_ref[2] += jnp.sum(mask0.astype(jnp.float32))
    acc_ref[3] += jnp.sum(jnp.where(mask1, nl1, 0.0))
    acc_ref[4] += jnp.sum(mask1.astype(jnp.float32))

    @pl.when(p == grid_m - 1)
    def _fin():
        pos_loss = acc_ref[0] / n_valid
        neg0 = acc_ref[1] / jnp.maximum(acc_ref[2], 1.0)
        neg1 = acc_ref[3] / jnp.maximum(acc_ref[4], 1.0)
        out_ref[0, 0] = pos_loss + (neg0 + neg1) / 2.0


def _tc_loss(posF0, posF1, subF0, subF1, pi0, pi1, ck0, ck1, cnts, n_valid):
    n_sub = subF0.shape[0]
    grid_m = M_PAD // BLK_M
    kern = functools.partial(
        _tc_loss_kernel, n_valid=n_valid, n_sub=n_sub, grid_m=grid_m)
    full = lambda shape: pl.BlockSpec(shape, lambda p: (0, 0))
    out = pl.pallas_call(
        kern,
        grid=(grid_m,),
        in_specs=[
            pl.BlockSpec((BLK_M, 128), lambda p: (p, 0)),
            pl.BlockSpec((BLK_M, 128), lambda p: (p, 0)),
            full((n_sub, 128)),
            full((n_sub, 128)),
            pl.BlockSpec((BLK_M, 1), lambda p: (p, 0)),
            pl.BlockSpec((BLK_M, 1), lambda p: (p, 0)),
            full((1, CK_PAD)),
            full((1, CK_PAD)),
            pl.BlockSpec(memory_space=pltpu.SMEM),
        ],
        out_specs=pl.BlockSpec(memory_space=pltpu.SMEM),
        out_shape=jax.ShapeDtypeStruct((1, 1), jnp.float32),
        scratch_shapes=[
            pltpu.SMEM((8,), jnp.float32),
            pltpu.VMEM((n_sub, 130), jnp.bfloat16),
            pltpu.VMEM((n_sub, 130), jnp.bfloat16),
            pltpu.VMEM((BLK_M, 1), jnp.int32),
            pltpu.VMEM((BLK_M, 1), jnp.int32),
        ],
        compiler_params=pltpu.CompilerParams(
            dimension_semantics=("arbitrary",)),
    )(posF0, posF1, subF0, subF1, pi0, pi1, ck0, ck1, cnts)
    return out[0, 0]


def kernel(F0, F1, matches):
    N0, N1 = int(F0.shape[0]), int(F1.shape[0])
    n_pairs = int(matches.shape[0])
    sel0, sel1, pos_sel = _selections(N0, N1, n_pairs)
    n_valid = len(pos_sel)
    n_sub = len(sel0)

    pos_pad = np.zeros(M_PAD, np.int32)
    pos_pad[:n_valid] = pos_sel
    off0 = (2 * pos_pad).reshape(NW, NPCH, PCH)
    off1 = (2 * pos_pad + 1).reshape(NW, NPCH, PCH)
    s0w = sel0.reshape(NW, n_sub // NW)
    s1w = sel1.reshape(NW, n_sub // NW)
    invT = np.full((2, N0), -1, np.int32)
    invT[0, sel1] = np.arange(n_sub, dtype=np.int32)
    invT[1, sel0] = np.arange(n_sub, dtype=np.int32)

    matches = matches.astype(jnp.int32)
    mflat = matches.reshape(-1)

    (posF0, posF1, subF0, subF1, pi0, pi1,
     ck0, ck1, cnts) = _sc_gather_fn(n_sub, n_pairs)(
        F0, F1, mflat,
        jnp.asarray(off0), jnp.asarray(off1),
        jnp.asarray(s0w), jnp.asarray(s1w), jnp.asarray(invT))

    pi0 = pi0.reshape(M_PAD, 1)
    pi1 = pi1.reshape(M_PAD, 1)
    ck0 = ck0.reshape(1, CK_PAD)
    ck1 = ck1.reshape(1, CK_PAD)

    return _tc_loss(posF0, posF1, subF0, subF1, pi0, pi1, ck0, ck1,
                    cnts, n_valid)
